# Initial kernel scaffold; baseline (speedup 1.0000x reference)
import jax
import jax.numpy as jnp
from jax import lax
from jax.experimental import pallas as pl
from jax.experimental.pallas import tpu as pltpu
from jax.experimental.pallas import tpu_sc as plsc

N = 10000
E = 320000
IN_DIM = 128
HEADS = 8
HI = 16
HID = HEADS * HI
OUT = 16

NC = 2
NS = 16
NW = NC * NS

CHUNK = 128
ETOT = E + N
CPW = -(-ETOT // (NW * CHUNK))
EPAD = NW * CPW * CHUNK
ROWS_PER_TILE = 640
NPAD = NS * ROWS_PER_TILE
ZCHUNKS = ROWS_PER_TILE // CHUNK


def _proj1_body(x_ref, w1_ref, ab_ref, h1_ref, asd_ref):
  h = jnp.dot(x_ref[...], w1_ref[...], preferred_element_type=jnp.float32)
  h1_ref[...] = h
  asd_ref[...] = jnp.dot(h, ab_ref[...], preferred_element_type=jnp.float32)


def _proj1(x, W1, AB, bn=1000):
  grid = (N // bn,)
  return pl.pallas_call(
      _proj1_body,
      grid=grid,
      in_specs=[
          pl.BlockSpec((bn, IN_DIM), lambda i: (i, 0)),
          pl.BlockSpec((IN_DIM, HID), lambda i: (0, 0)),
          pl.BlockSpec((HID, 16), lambda i: (0, 0)),
      ],
      out_specs=[
          pl.BlockSpec((bn, HID), lambda i: (i, 0)),
          pl.BlockSpec((bn, 16), lambda i: (i, 0)),
      ],
      out_shape=[
          jax.ShapeDtypeStruct((N, HID), jnp.float32),
          jax.ShapeDtypeStruct((N, 16), jnp.float32),
      ],
  )(x, W1, AB)


def _edge1_body(h1, asd1, srcs, dsts, acc_out, den_out,
                idx_src, idx_dst, hbuf, asbuf, adbuf, wbuf, zbuf,
                s0, s1, s2, acc_sh, den_sh):
  c = lax.axis_index("c")
  s = lax.axis_index("s")
  gw = c * NS + s

  def zrow(r, _):
    for j in range(HID // 16):
      zbuf[r, pl.ds(16 * j, 16)] = jnp.zeros((16,), jnp.float32)
    return 0
  lax.fori_loop(0, CHUNK, zrow, 0)
  base_rows = s * ROWS_PER_TILE
  for k in range(ZCHUNKS):
    pltpu.sync_copy(zbuf, acc_sh.at[pl.ds(base_rows + k * CHUNK, CHUNK)])
    pltpu.sync_copy(zbuf.at[:, pl.ds(0, 16)],
                    den_sh.at[pl.ds(base_rows + k * CHUNK, CHUNK)])
  plsc.subcore_barrier()

  perm = (lax.iota(jnp.int32, 16) & 7) + 8

  def chunk_body(i, _):
    base = pl.multiple_of((gw * CPW + i) * CHUNK, CHUNK)
    pltpu.sync_copy(srcs.at[pl.ds(base, CHUNK)], idx_src)
    pltpu.sync_copy(dsts.at[pl.ds(base, CHUNK)], idx_dst)
    cp0 = pltpu.async_copy(h1.at[idx_src], hbuf, s0)
    cp1 = pltpu.async_copy(asd1.at[idx_src], asbuf, s1)
    cp2 = pltpu.async_copy(asd1.at[idx_dst], adbuf, s2)
    cp0.wait()
    cp1.wait()
    cp2.wait()

    def edge_body(e, _):
      e16 = jnp.full((16,), e, jnp.int32)
      arow = asbuf[e, :]
      dperm = plsc.load_gather(adbuf, [e16, perm])
      alpha = arow + dperm
      w16 = jnp.exp(jnp.maximum(alpha, 0.2 * alpha))
      wbuf[e, :] = w16
      for j in range(HEADS):
        wb = plsc.load_gather(wbuf, [e16, jnp.full((16,), j, jnp.int32)])
        hbuf[e, pl.ds(16 * j, 16)] = hbuf[e, pl.ds(16 * j, 16)] * wb
      return 0
    lax.fori_loop(0, CHUNK, edge_body, 0)

    pltpu.sync_copy(hbuf, acc_sh.at[idx_dst], add=True)
    pltpu.sync_copy(wbuf, den_sh.at[idx_dst], add=True)
    return 0

  lax.fori_loop(0, CPW, chunk_body, 0)
  plsc.subcore_barrier()

  for k in range(ZCHUNKS):
    r0 = base_rows + k * CHUNK
    pltpu.sync_copy(acc_sh.at[pl.ds(r0, CHUNK)], acc_out.at[c, pl.ds(r0, CHUNK)])
    pltpu.sync_copy(den_sh.at[pl.ds(r0, CHUNK)], den_out.at[c, pl.ds(r0, CHUNK)])


def _edge1(h1, asd1, srcs, dsts):
  mesh = plsc.VectorSubcoreMesh(core_axis_name="c", subcore_axis_name="s",
                                num_cores=NC, num_subcores=NS)
  fn = pl.kernel(
      _edge1_body,
      out_type=[
          jax.ShapeDtypeStruct((NC, NPAD, HID), jnp.float32),
          jax.ShapeDtypeStruct((NC, NPAD, 16), jnp.float32),
      ],
      mesh=mesh,
      scratch_types=[
          pltpu.VMEM((CHUNK,), jnp.int32),
          pltpu.VMEM((CHUNK,), jnp.int32),
          pltpu.VMEM((CHUNK, HID), jnp.float32),
          pltpu.VMEM((CHUNK, 16), jnp.float32),
          pltpu.VMEM((CHUNK, 16), jnp.float32),
          pltpu.VMEM((CHUNK, 16), jnp.float32),
          pltpu.VMEM((CHUNK, HID), jnp.float32),
          pltpu.SemaphoreType.DMA,
          pltpu.SemaphoreType.DMA,
          pltpu.SemaphoreType.DMA,
          pltpu.VMEM_SHARED((NPAD, HID), jnp.float32),
          pltpu.VMEM_SHARED((NPAD, 16), jnp.float32),
      ],
  )
  return fn(h1, asd1, srcs, dsts)


def _comb1_body(accA, accB, denA, denB, w2_ref, s_ref, g_ref, g2_ref,
                hs2_ref, ad2_ref):
  den128 = jnp.dot(denA[...] + denB[...], s_ref[...],
                   preferred_element_type=jnp.float32) + 1e-16
  h2 = jnp.maximum((accA[...] + accB[...]) / den128, 0.0)
  f2 = jnp.dot(h2, w2_ref[...], preferred_element_type=jnp.float32)
  hs2_ref[...] = jnp.dot(f2, g_ref[...], preferred_element_type=jnp.float32)
  ad2_ref[...] = jnp.dot(f2, g2_ref[...], preferred_element_type=jnp.float32)


def _comb1(accA, accB, denA, denB, W2, S, G, G2, bn=1000):
  grid = (N // bn,)
  return pl.pallas_call(
      _comb1_body,
      grid=grid,
      in_specs=[
          pl.BlockSpec((bn, HID), lambda i: (i, 0)),
          pl.BlockSpec((bn, HID), lambda i: (i, 0)),
          pl.BlockSpec((bn, 16), lambda i: (i, 0)),
          pl.BlockSpec((bn, 16), lambda i: (i, 0)),
          pl.BlockSpec((HID, OUT), lambda i: (0, 0)),
          pl.BlockSpec((16, HID), lambda i: (0, 0)),
          pl.BlockSpec((OUT, 32), lambda i: (0, 0)),
          pl.BlockSpec((OUT, 16), lambda i: (0, 0)),
      ],
      out_specs=[
          pl.BlockSpec((bn, 32), lambda i: (i, 0)),
          pl.BlockSpec((bn, 16), lambda i: (i, 0)),
      ],
      out_shape=[
          jax.ShapeDtypeStruct((N, 32), jnp.float32),
          jax.ShapeDtypeStruct((N, 16), jnp.float32),
      ],
  )(accA, accB, denA, denB, W2, S, G, G2)


def _edge2_body(hs2, ad2, srcs, dsts, acc_out, den_out,
                idx_src, idx_dst, sbuf, dbuf, mbuf, wbuf, zbuf,
                s0, s1, acc_sh, den_sh):
  c = lax.axis_index("c")
  s = lax.axis_index("s")
  gw = c * NS + s

  def zrow(r, _):
    zbuf[r, :] = jnp.zeros((16,), jnp.float32)
    return 0
  lax.fori_loop(0, CHUNK, zrow, 0)
  base_rows = s * ROWS_PER_TILE
  for k in range(ZCHUNKS):
    pltpu.sync_copy(zbuf, acc_sh.at[pl.ds(base_rows + k * CHUNK, CHUNK)])
    pltpu.sync_copy(zbuf, den_sh.at[pl.ds(base_rows + k * CHUNK, CHUNK)])
  plsc.subcore_barrier()

  zero16 = jnp.zeros((16,), jnp.int32)

  def chunk_body(i, _):
    base = pl.multiple_of((gw * CPW + i) * CHUNK, CHUNK)
    pltpu.sync_copy(srcs.at[pl.ds(base, CHUNK)], idx_src)
    pltpu.sync_copy(dsts.at[pl.ds(base, CHUNK)], idx_dst)
    cp0 = pltpu.async_copy(hs2.at[idx_src], sbuf, s0)
    cp1 = pltpu.async_copy(ad2.at[idx_dst], dbuf, s1)
    cp0.wait()
    cp1.wait()

    def edge_body(e, _):
      e16 = jnp.full((16,), e, jnp.int32)
      alpha = sbuf[e, pl.ds(16, 16)] + dbuf[e, :]
      w16 = jnp.exp(jnp.maximum(alpha, 0.2 * alpha))
      wbuf[e, :] = w16
      wb = plsc.load_gather(wbuf, [e16, zero16])
      mbuf[e, :] = sbuf[e, pl.ds(0, 16)] * wb
      wbuf[e, :] = wb
      return 0
    lax.fori_loop(0, CHUNK, edge_body, 0)

    pltpu.sync_copy(mbuf, acc_sh.at[idx_dst], add=True)
    pltpu.sync_copy(wbuf, den_sh.at[idx_dst], add=True)
    return 0

  lax.fori_loop(0, CPW, chunk_body, 0)
  plsc.subcore_barrier()

  for k in range(ZCHUNKS):
    r0 = base_rows + k * CHUNK
    pltpu.sync_copy(acc_sh.at[pl.ds(r0, CHUNK)], acc_out.at[c, pl.ds(r0, CHUNK)])
    pltpu.sync_copy(den_sh.at[pl.ds(r0, CHUNK)], den_out.at[c, pl.ds(r0, CHUNK)])


def _edge2(hs2, ad2, srcs, dsts):
  mesh = plsc.VectorSubcoreMesh(core_axis_name="c", subcore_axis_name="s",
                                num_cores=NC, num_subcores=NS)
  fn = pl.kernel(
      _edge2_body,
      out_type=[
          jax.ShapeDtypeStruct((NC, NPAD, 16), jnp.float32),
          jax.ShapeDtypeStruct((NC, NPAD, 16), jnp.float32),
      ],
      mesh=mesh,
      scratch_types=[
          pltpu.VMEM((CHUNK,), jnp.int32),
          pltpu.VMEM((CHUNK,), jnp.int32),
          pltpu.VMEM((CHUNK, 32), jnp.float32),
          pltpu.VMEM((CHUNK, 16), jnp.float32),
          pltpu.VMEM((CHUNK, 16), jnp.float32),
          pltpu.VMEM((CHUNK, 16), jnp.float32),
          pltpu.VMEM((CHUNK, 16), jnp.float32),
          pltpu.SemaphoreType.DMA,
          pltpu.SemaphoreType.DMA,
          pltpu.VMEM_SHARED((NPAD, 16), jnp.float32),
          pltpu.VMEM_SHARED((NPAD, 16), jnp.float32),
      ],
  )
  return fn(hs2, ad2, srcs, dsts)


def _final_body(a2A, a2B, d2A, d2B, b2_ref, out_ref):
  o = (a2A[...] + a2B[...]) / (d2A[...] + d2B[...] + 1e-16) + b2_ref[...]
  m = jnp.max(o, axis=1, keepdims=True)
  ex = jnp.exp(o - m)
  out_ref[...] = (o - m) - jnp.log(jnp.sum(ex, axis=1, keepdims=True))


def _final(a2A, a2B, d2A, d2B, b2, bn=1000):
  grid = (N // bn,)
  return pl.pallas_call(
      _final_body,
      grid=grid,
      in_specs=[
          pl.BlockSpec((bn, 16), lambda i: (i, 0)),
          pl.BlockSpec((bn, 16), lambda i: (i, 0)),
          pl.BlockSpec((bn, 16), lambda i: (i, 0)),
          pl.BlockSpec((bn, 16), lambda i: (i, 0)),
          pl.BlockSpec((1, 16), lambda i: (0, 0)),
      ],
      out_specs=pl.BlockSpec((bn, 16), lambda i: (i, 0)),
      out_shape=jax.ShapeDtypeStruct((N, 16), jnp.float32),
  )(a2A, a2B, d2A, d2B, b2)


@jax.jit
def kernel(x, edge_index, W1, att_src1, att_dst1, b1, W2, att_src2,
           att_dst2, b2):
  eye8 = jnp.eye(HEADS, dtype=jnp.float32)
  ab_src = (att_src1[:, :, None] * eye8[:, None, :]).reshape(HID, HEADS)
  ab_dst = (att_dst1[:, :, None] * eye8[:, None, :]).reshape(HID, HEADS)
  AB = jnp.concatenate([ab_src, ab_dst], axis=1)

  rows = jnp.arange(16)
  cols = jnp.arange(HID)
  S = jnp.where((rows[:, None] == cols[None, :] // HI) & (rows[:, None] < 8),
                1.0, 0.0).astype(jnp.float32)

  G = jnp.zeros((OUT, 32), jnp.float32)
  G = G.at[:, :OUT].set(jnp.eye(OUT, dtype=jnp.float32))
  G = G.at[:, OUT:].set(jnp.broadcast_to(att_src2[0][:, None], (OUT, 16)))
  G2 = jnp.broadcast_to(att_dst2[0][:, None], (OUT, 16)).astype(jnp.float32)

  loops = jnp.arange(N, dtype=jnp.int32)
  pad = EPAD - ETOT
  srcs = jnp.concatenate([edge_index[0], loops,
                          jnp.zeros((pad,), jnp.int32)])
  dsts = jnp.concatenate([edge_index[1], loops,
                          jnp.full((pad,), N, jnp.int32)])

  h1, asd1 = _proj1(x, W1, AB)
  accs, dens = _edge1(h1, asd1, srcs, dsts)
  hs2, ad2 = _comb1(accs[0, :N], accs[1, :N], dens[0, :N], dens[1, :N],
                    W2, S, G, G2)
  acc2, den2 = _edge2(hs2, ad2, srcs, dsts)
  out = _final(acc2[0, :N], acc2[1, :N], den2[0, :N], den2[1, :N],
               b2.reshape(1, OUT))
  return out


# speedup vs baseline: 40.8684x; 40.8684x over previous
import jax
import jax.numpy as jnp
from jax import lax
from jax.experimental import pallas as pl
from jax.experimental.pallas import tpu as pltpu
from jax.experimental.pallas import tpu_sc as plsc

N = 10000
E = 320000
IN_DIM = 128
HEADS = 8
HI = 16
HID = HEADS * HI
OUT = 16

NC = 2
NS = 16
NW = NC * NS

CHUNK = 128
ETOT = E + N
CPW = -(-ETOT // (NW * CHUNK))
EPAD = NW * CPW * CHUNK
CPT = EPAD // (NS * CHUNK)
ROWS_PER_TILE = 626
NPAD = NS * ROWS_PER_TILE
ZSLICES = [128, 128, 128, 128, 114]
EPC = EPAD // NC


def _proj1_body(x_ref, w1_ref, att_ref, h1a_ref, h1b_ref, attout_ref):
  h = jnp.dot(x_ref[...], w1_ref[...], preferred_element_type=jnp.float32)
  h1a_ref[...] = h[:, :HID // 2]
  h1b_ref[...] = h[:, HID // 2:]
  attout_ref[...] = jnp.dot(h, att_ref[...],
                            preferred_element_type=jnp.float32)


def _proj1(x, W1, ATT, bn=1000):
  grid = (N // bn,)
  return pl.pallas_call(
      _proj1_body,
      grid=grid,
      in_specs=[
          pl.BlockSpec((bn, IN_DIM), lambda i: (i, 0)),
          pl.BlockSpec((IN_DIM, HID), lambda i: (0, 0)),
          pl.BlockSpec((HID, 64), lambda i: (0, 0)),
      ],
      out_specs=[
          pl.BlockSpec((bn, HID // 2), lambda i: (i, 0)),
          pl.BlockSpec((bn, HID // 2), lambda i: (i, 0)),
          pl.BlockSpec((bn, 64), lambda i: (i, 0)),
      ],
      out_shape=[
          jax.ShapeDtypeStruct((N, HID // 2), jnp.float32),
          jax.ShapeDtypeStruct((N, HID // 2), jnp.float32),
          jax.ShapeDtypeStruct((N, 64), jnp.float32),
      ],
  )(x, W1, ATT)


def _edge1_body(hstk, asd1, ads1, edges, acc_out, den_out,
                idx_pk, idx_src, idx_dst, idx_dstg, hbuf, asbuf, adbuf, wbuf,
                zbuf, zbuf16, s0, s1, s2, acc_sh, den_sh):
  c = lax.axis_index("c")
  s = lax.axis_index("s")
  HW = HID // 2

  def zrow(r, _):
    for j in range(HW // 16):
      zbuf[r, pl.ds(16 * j, 16)] = jnp.zeros((16,), jnp.float32)
    zbuf16[r, :] = jnp.zeros((16,), jnp.float32)
    return 0
  lax.fori_loop(0, CHUNK, zrow, 0)
  base_rows = s * ROWS_PER_TILE
  r0 = base_rows
  for zr in ZSLICES:
    pltpu.sync_copy(zbuf.at[pl.ds(0, zr)], acc_sh.at[pl.ds(r0, zr)])
    pltpu.sync_copy(zbuf16.at[pl.ds(0, zr)], den_sh.at[pl.ds(r0, zr)])
    r0 += zr
  plsc.subcore_barrier()

  coff = c * N

  def chunk_body(i, _):
    base = pl.multiple_of((s * CPT + i) * CHUNK, CHUNK)
    pltpu.async_copy(edges.at[pl.ds(base, CHUNK)], idx_pk, s0).wait()
    def unpack(q, _):
      v = idx_pk[pl.ds(q * 16, 16)]
      idx_src[pl.ds(q * 16, 16)] = (v & 16383) + coff
      d = v >> 14
      idx_dst[pl.ds(q * 16, 16)] = d
      idx_dstg[pl.ds(q * 16, 16)] = d + coff
      return 0
    lax.fori_loop(0, CHUNK // 16, unpack, 0)
    cp0 = pltpu.async_copy(hstk.at[idx_src], hbuf, s0)
    cp1 = pltpu.async_copy(asd1.at[idx_src], asbuf, s1)
    cp2 = pltpu.async_copy(ads1.at[idx_dstg], adbuf, s2)
    cp0.wait()
    cp1.wait()
    cp2.wait()

    def edge_body(e, _):
      alpha = asbuf[e, :] + adbuf[e, :]
      w16 = jnp.exp(jnp.maximum(alpha, 0.2 * alpha))
      wbuf[e, :] = w16
      for j in range(HEADS // 2):
        wb = jnp.full((16,), w16[j], jnp.float32)
        hbuf[e, pl.ds(16 * j, 16)] = hbuf[e, pl.ds(16 * j, 16)] * wb
      return 0
    lax.fori_loop(0, CHUNK, edge_body, 0)

    pltpu.sync_copy(hbuf, acc_sh.at[idx_dst], add=True)
    pltpu.sync_copy(wbuf, den_sh.at[idx_dst], add=True)
    return 0

  lax.fori_loop(0, CPT, chunk_body, 0)
  plsc.subcore_barrier()

  r0 = base_rows
  for zr in ZSLICES:
    pltpu.sync_copy(acc_sh.at[pl.ds(r0, zr)], acc_out.at[c, pl.ds(r0, zr)])
    pltpu.sync_copy(den_sh.at[pl.ds(r0, zr)], den_out.at[c, pl.ds(r0, zr)])
    r0 += zr


def _edge1(hstk, asd1, ads1, edges):
  mesh = plsc.VectorSubcoreMesh(core_axis_name="c", subcore_axis_name="s",
                                num_cores=NC, num_subcores=NS)
  HW = HID // 2
  fn = pl.kernel(
      _edge1_body,
      out_type=[
          jax.ShapeDtypeStruct((NC, NPAD, HW), jnp.float32),
          jax.ShapeDtypeStruct((NC, NPAD, 16), jnp.float32),
      ],
      mesh=mesh,
      scratch_types=[
          pltpu.VMEM((CHUNK,), jnp.int32),
          pltpu.VMEM((CHUNK,), jnp.int32),
          pltpu.VMEM((CHUNK,), jnp.int32),
          pltpu.VMEM((CHUNK,), jnp.int32),
          pltpu.VMEM((CHUNK, HW), jnp.float32),
          pltpu.VMEM((CHUNK, 16), jnp.float32),
          pltpu.VMEM((CHUNK, 16), jnp.float32),
          pltpu.VMEM((CHUNK, 16), jnp.float32),
          pltpu.VMEM((CHUNK, HW), jnp.float32),
          pltpu.VMEM((CHUNK, 16), jnp.float32),
          pltpu.SemaphoreType.DMA,
          pltpu.SemaphoreType.DMA,
          pltpu.SemaphoreType.DMA,
          pltpu.VMEM_SHARED((NPAD, HW), jnp.float32),
          pltpu.VMEM_SHARED((NPAD, 16), jnp.float32),
      ],
      compiler_params=pltpu.CompilerParams(use_tc_tiling_on_sc=False),
  )
  return fn(hstk, asd1, ads1, edges)


def _comb1_body(accA, accB, denA, denB, w2a_ref, w2b_ref, sl_ref, sh_ref,
                g_ref, g2_ref, b1a_ref, b1b_ref, hs2_ref, ad2_ref):
  dA = jnp.dot(denA[...], sl_ref[...],
               preferred_element_type=jnp.float32) + 1e-16
  dB = jnp.dot(denB[...], sh_ref[...],
               preferred_element_type=jnp.float32) + 1e-16
  hA = jnp.maximum(accA[...] / dA + b1a_ref[...], 0.0)
  hB = jnp.maximum(accB[...] / dB + b1b_ref[...], 0.0)
  f2 = (jnp.dot(hA, w2a_ref[...], preferred_element_type=jnp.float32)
        + jnp.dot(hB, w2b_ref[...], preferred_element_type=jnp.float32))
  hs2_ref[...] = jnp.dot(f2, g_ref[...], preferred_element_type=jnp.float32)
  ad2_ref[...] = jnp.dot(f2, g2_ref[...], preferred_element_type=jnp.float32)


def _comb1(accA, accB, denA, denB, W2a, W2b, SL, SH, G, G2, b1a, b1b,
           bn=1000):
  grid = (N // bn,)
  HW = HID // 2
  return pl.pallas_call(
      _comb1_body,
      grid=grid,
      in_specs=[
          pl.BlockSpec((bn, HW), lambda i: (i, 0)),
          pl.BlockSpec((bn, HW), lambda i: (i, 0)),
          pl.BlockSpec((bn, 16), lambda i: (i, 0)),
          pl.BlockSpec((bn, 16), lambda i: (i, 0)),
          pl.BlockSpec((HW, OUT), lambda i: (0, 0)),
          pl.BlockSpec((HW, OUT), lambda i: (0, 0)),
          pl.BlockSpec((16, HW), lambda i: (0, 0)),
          pl.BlockSpec((16, HW), lambda i: (0, 0)),
          pl.BlockSpec((OUT, 32), lambda i: (0, 0)),
          pl.BlockSpec((OUT, 16), lambda i: (0, 0)),
          pl.BlockSpec((1, HW), lambda i: (0, 0)),
          pl.BlockSpec((1, HW), lambda i: (0, 0)),
      ],
      out_specs=[
          pl.BlockSpec((bn, 32), lambda i: (i, 0)),
          pl.BlockSpec((bn, 16), lambda i: (i, 0)),
      ],
      out_shape=[
          jax.ShapeDtypeStruct((N, 32), jnp.float32),
          jax.ShapeDtypeStruct((N, 16), jnp.float32),
      ],
  )(accA, accB, denA, denB, W2a, W2b, SL, SH, G, G2, b1a, b1b)


def _edge2_body(hs2, ad2, edges, acc_out, den_out,
                idx_pk, idx_src, idx_dst, sbuf, dbuf, mbuf, wbuf, zbuf,
                s0, s1, acc_sh, den_sh):
  c = lax.axis_index("c")
  s = lax.axis_index("s")
  gw = c * NS + s

  def zrow(r, _):
    zbuf[r, :] = jnp.zeros((16,), jnp.float32)
    return 0
  lax.fori_loop(0, CHUNK, zrow, 0)
  base_rows = s * ROWS_PER_TILE
  r0 = base_rows
  for zr in ZSLICES:
    pltpu.sync_copy(zbuf.at[pl.ds(0, zr)], acc_sh.at[pl.ds(r0, zr)])
    pltpu.sync_copy(zbuf.at[pl.ds(0, zr)], den_sh.at[pl.ds(r0, zr)])
    r0 += zr
  plsc.subcore_barrier()

  zero16 = jnp.zeros((16,), jnp.int32)

  def chunk_body(i, _):
    base = pl.multiple_of(((c * NS + s) * CPW + i) * CHUNK, CHUNK)
    pltpu.async_copy(edges.at[pl.ds(base, CHUNK)], idx_pk, s0).wait()
    def unpack(q, _):
      v = idx_pk[pl.ds(q * 16, 16)]
      idx_src[pl.ds(q * 16, 16)] = v & 16383
      idx_dst[pl.ds(q * 16, 16)] = v >> 14
      return 0
    lax.fori_loop(0, CHUNK // 16, unpack, 0)
    cp0 = pltpu.async_copy(hs2.at[idx_src], sbuf, s0)
    cp1 = pltpu.async_copy(ad2.at[idx_dst], dbuf, s1)
    cp0.wait()
    cp1.wait()

    def edge_body(e, _):
      alpha = sbuf[e, pl.ds(16, 16)] + dbuf[e, :]
      w16 = jnp.exp(jnp.maximum(alpha, 0.2 * alpha))
      wb = jnp.full((16,), w16[0], jnp.float32)
      mbuf[e, :] = sbuf[e, pl.ds(0, 16)] * wb
      wbuf[e, :] = wb
      return 0
    lax.fori_loop(0, CHUNK, edge_body, 0)

    pltpu.sync_copy(mbuf, acc_sh.at[idx_dst], add=True)
    pltpu.sync_copy(wbuf, den_sh.at[idx_dst], add=True)
    return 0

  lax.fori_loop(0, CPW, chunk_body, 0)
  plsc.subcore_barrier()

  r0 = base_rows
  for zr in ZSLICES:
    pltpu.sync_copy(acc_sh.at[pl.ds(r0, zr)], acc_out.at[c, pl.ds(r0, zr)])
    pltpu.sync_copy(den_sh.at[pl.ds(r0, zr)], den_out.at[c, pl.ds(r0, zr)])
    r0 += zr


def _edge2(hs2, ad2, edges):
  mesh = plsc.VectorSubcoreMesh(core_axis_name="c", subcore_axis_name="s",
                                num_cores=NC, num_subcores=NS)
  fn = pl.kernel(
      _edge2_body,
      out_type=[
          jax.ShapeDtypeStruct((NC, NPAD, 16), jnp.float32),
          jax.ShapeDtypeStruct((NC, NPAD, 16), jnp.float32),
      ],
      mesh=mesh,
      scratch_types=[
          pltpu.VMEM((CHUNK,), jnp.int32),
          pltpu.VMEM((CHUNK,), jnp.int32),
          pltpu.VMEM((CHUNK,), jnp.int32),
          pltpu.VMEM((CHUNK, 32), jnp.float32),
          pltpu.VMEM((CHUNK, 16), jnp.float32),
          pltpu.VMEM((CHUNK, 16), jnp.float32),
          pltpu.VMEM((CHUNK, 16), jnp.float32),
          pltpu.VMEM((CHUNK, 16), jnp.float32),
          pltpu.SemaphoreType.DMA,
          pltpu.SemaphoreType.DMA,
          pltpu.VMEM_SHARED((NPAD, 16), jnp.float32),
          pltpu.VMEM_SHARED((NPAD, 16), jnp.float32),
      ],
      compiler_params=pltpu.CompilerParams(use_tc_tiling_on_sc=False),
  )
  return fn(hs2, ad2, edges)


def _final_body(a2A, a2B, d2A, d2B, b2_ref, out_ref):
  o = (a2A[...] + a2B[...]) / (d2A[...] + d2B[...] + 1e-16) + b2_ref[...]
  m = jnp.max(o, axis=1, keepdims=True)
  ex = jnp.exp(o - m)
  out_ref[...] = (o - m) - jnp.log(jnp.sum(ex, axis=1, keepdims=True))


def _final(a2A, a2B, d2A, d2B, b2, bn=1000):
  grid = (N // bn,)
  return pl.pallas_call(
      _final_body,
      grid=grid,
      in_specs=[
          pl.BlockSpec((bn, 16), lambda i: (i, 0)),
          pl.BlockSpec((bn, 16), lambda i: (i, 0)),
          pl.BlockSpec((bn, 16), lambda i: (i, 0)),
          pl.BlockSpec((bn, 16), lambda i: (i, 0)),
          pl.BlockSpec((1, 16), lambda i: (0, 0)),
      ],
      out_specs=pl.BlockSpec((bn, 16), lambda i: (i, 0)),
      out_shape=jax.ShapeDtypeStruct((N, 16), jnp.float32),
  )(a2A, a2B, d2A, d2B, b2)


@jax.jit
def kernel(x, edge_index, W1, att_src1, att_dst1, b1, W2, att_src2,
           att_dst2, b2):
  eye8 = jnp.eye(HEADS, dtype=jnp.float32)
  ab_src = (att_src1[:, :, None] * eye8[:, None, :]).reshape(HID, HEADS)
  ab_dst = (att_dst1[:, :, None] * eye8[:, None, :]).reshape(HID, HEADS)
  AB = jnp.concatenate([ab_src, ab_dst], axis=1)
  AB2 = jnp.concatenate([ab_dst, ab_src], axis=1)
  perm16 = jnp.array([4, 5, 6, 7, 0, 1, 2, 3,
                      12, 13, 14, 15, 8, 9, 10, 11])
  ATT = jnp.concatenate([AB, AB[:, perm16], AB2, AB2[:, perm16]], axis=1)

  rows = jnp.arange(16)
  cols = jnp.arange(HID // 2)
  SL = (rows[:, None] == cols[None, :] // HI).astype(jnp.float32)

  G = jnp.zeros((OUT, 32), jnp.float32)
  G = G.at[:, :OUT].set(jnp.eye(OUT, dtype=jnp.float32))
  G = G.at[:, OUT:].set(jnp.broadcast_to(att_src2[0][:, None], (OUT, 16)))
  G2 = jnp.broadcast_to(att_dst2[0][:, None], (OUT, 16)).astype(jnp.float32)

  loops = jnp.arange(N, dtype=jnp.int32)
  pad = EPAD - ETOT
  srcs = jnp.concatenate([edge_index[0], loops,
                          jnp.zeros((pad,), jnp.int32)])
  dsts = jnp.concatenate([edge_index[1], loops,
                          jnp.full((pad,), N, jnp.int32)])
  edges = srcs | (dsts << 14)

  h1a, h1b, attout = _proj1(x, W1, ATT)
  hstk = jnp.concatenate([h1a, h1b], axis=0)
  asd_stk = jnp.concatenate([attout[:, 0:16], attout[:, 16:32]], axis=0)
  ads_stk = jnp.concatenate([attout[:, 32:48], attout[:, 48:64]], axis=0)
  accs, dens = _edge1(hstk, asd_stk, ads_stk, edges)
  hs2, ad2 = _comb1(accs[0, :N], accs[1, :N], dens[0, :N], dens[1, :N],
                    W2[:HID // 2], W2[HID // 2:], SL, SL, G, G2,
                    b1[:HID // 2].reshape(1, -1), b1[HID // 2:].reshape(1, -1))
  acc2, den2 = _edge2(hs2, ad2, edges)
  out = _final(acc2[0, :N], acc2[1, :N], den2[0, :N], den2[1, :N],
               b2.reshape(1, OUT))
  return out


# speedup vs baseline: 41.5942x; 1.0178x over previous
import jax
import jax.numpy as jnp
from jax import lax
from jax.experimental import pallas as pl
from jax.experimental.pallas import tpu as pltpu
from jax.experimental.pallas import tpu_sc as plsc

N = 10000
E = 320000
IN_DIM = 128
HEADS = 8
HI = 16
HID = HEADS * HI
OUT = 16

NC = 2
NS = 16
NW = NC * NS

CHUNK = 128
ETOT = E + N
CPW = -(-ETOT // (NW * CHUNK))
EPAD = NW * CPW * CHUNK
CPT = EPAD // (NS * CHUNK)
ROWS_PER_TILE = 626
NPAD = NS * ROWS_PER_TILE
ZSLICES = [128, 128, 128, 128, 114]
EPC = EPAD // NC


def _proj1_body(x_ref, w1_ref, att_ref, h1a_ref, h1b_ref, attout_ref):
  h = jnp.dot(x_ref[...], w1_ref[...], preferred_element_type=jnp.float32)
  h1a_ref[...] = h[:, :HID // 2]
  h1b_ref[...] = h[:, HID // 2:]
  attout_ref[...] = jnp.dot(h, att_ref[...],
                            preferred_element_type=jnp.float32)


def _proj1(x, W1, ATT, bn=1000):
  grid = (N // bn,)
  return pl.pallas_call(
      _proj1_body,
      grid=grid,
      in_specs=[
          pl.BlockSpec((bn, IN_DIM), lambda i: (i, 0)),
          pl.BlockSpec((IN_DIM, HID), lambda i: (0, 0)),
          pl.BlockSpec((HID, 64), lambda i: (0, 0)),
      ],
      out_specs=[
          pl.BlockSpec((bn, HID // 2), lambda i: (i, 0)),
          pl.BlockSpec((bn, HID // 2), lambda i: (i, 0)),
          pl.BlockSpec((bn, 64), lambda i: (i, 0)),
      ],
      out_shape=[
          jax.ShapeDtypeStruct((N, HID // 2), jnp.float32),
          jax.ShapeDtypeStruct((N, HID // 2), jnp.float32),
          jax.ShapeDtypeStruct((N, 64), jnp.float32),
      ],
  )(x, W1, ATT)


def _edge1_body(hstk, asd1, ads1, edges, acc_out, den_out,
                idx_pk, idx_src, idx_dst, idx_dstg, hbuf, asbuf, adbuf, wbuf,
                zbuf, zbuf16, s0, s1, s2, acc_sh, den_sh):
  c = lax.axis_index("c")
  s = lax.axis_index("s")
  HW = HID // 2

  def zrow(r, _):
    for j in range(HW // 16):
      zbuf[r, pl.ds(16 * j, 16)] = jnp.zeros((16,), jnp.float32)
    zbuf16[r, :] = jnp.zeros((16,), jnp.float32)
    return 0
  lax.fori_loop(0, CHUNK, zrow, 0)
  base_rows = s * ROWS_PER_TILE
  r0 = base_rows
  for zr in ZSLICES:
    pltpu.sync_copy(zbuf.at[pl.ds(0, zr)], acc_sh.at[pl.ds(r0, zr)])
    pltpu.sync_copy(zbuf16.at[pl.ds(0, zr)], den_sh.at[pl.ds(r0, zr)])
    r0 += zr
  plsc.subcore_barrier()

  coff = c * NPAD

  def chunk_body(i, _):
    base = pl.multiple_of((s * CPT + i) * CHUNK, CHUNK)
    pltpu.async_copy(edges.at[pl.ds(base, CHUNK)], idx_pk, s0).wait()
    def unpack(q, _):
      v = idx_pk[pl.ds(q * 16, 16)]
      idx_src[pl.ds(q * 16, 16)] = (v & 16383) + coff
      d = v >> 14
      idx_dst[pl.ds(q * 16, 16)] = d
      idx_dstg[pl.ds(q * 16, 16)] = d + coff
      return 0
    lax.fori_loop(0, CHUNK // 16, unpack, 0)
    cp0 = pltpu.async_copy(hstk.at[idx_src], hbuf, s0)
    cp1 = pltpu.async_copy(asd1.at[idx_src], asbuf, s1)
    cp2 = pltpu.async_copy(ads1.at[idx_dstg], adbuf, s2)
    cp0.wait()
    cp1.wait()
    cp2.wait()

    def edge_body(e, _):
      alpha = asbuf[e, :] + adbuf[e, :]
      w16 = jnp.exp(jnp.maximum(alpha, 0.2 * alpha))
      wbuf[e, :] = w16
      for j in range(HEADS // 2):
        wb = jnp.full((16,), w16[j], jnp.float32)
        hbuf[e, pl.ds(16 * j, 16)] = hbuf[e, pl.ds(16 * j, 16)] * wb
      return 0
    lax.fori_loop(0, CHUNK, edge_body, 0)

    pltpu.sync_copy(hbuf, acc_sh.at[idx_dst], add=True)
    pltpu.sync_copy(wbuf, den_sh.at[idx_dst], add=True)
    return 0

  lax.fori_loop(0, CPT, chunk_body, 0)
  plsc.subcore_barrier()

  r0 = base_rows
  for zr in ZSLICES:
    pltpu.sync_copy(acc_sh.at[pl.ds(r0, zr)], acc_out.at[c, pl.ds(r0, zr)])
    pltpu.sync_copy(den_sh.at[pl.ds(r0, zr)], den_out.at[c, pl.ds(r0, zr)])
    r0 += zr


def _edge1(hstk, asd1, ads1, edges):
  mesh = plsc.VectorSubcoreMesh(core_axis_name="c", subcore_axis_name="s",
                                num_cores=NC, num_subcores=NS)
  HW = HID // 2
  fn = pl.kernel(
      _edge1_body,
      out_type=[
          jax.ShapeDtypeStruct((NC, NPAD, HW), jnp.float32),
          jax.ShapeDtypeStruct((NC, NPAD, 16), jnp.float32),
      ],
      mesh=mesh,
      scratch_types=[
          pltpu.VMEM((CHUNK,), jnp.int32),
          pltpu.VMEM((CHUNK,), jnp.int32),
          pltpu.VMEM((CHUNK,), jnp.int32),
          pltpu.VMEM((CHUNK,), jnp.int32),
          pltpu.VMEM((CHUNK, HW), jnp.float32),
          pltpu.VMEM((CHUNK, 16), jnp.float32),
          pltpu.VMEM((CHUNK, 16), jnp.float32),
          pltpu.VMEM((CHUNK, 16), jnp.float32),
          pltpu.VMEM((CHUNK, HW), jnp.float32),
          pltpu.VMEM((CHUNK, 16), jnp.float32),
          pltpu.SemaphoreType.DMA,
          pltpu.SemaphoreType.DMA,
          pltpu.SemaphoreType.DMA,
          pltpu.VMEM_SHARED((NPAD, HW), jnp.float32),
          pltpu.VMEM_SHARED((NPAD, 16), jnp.float32),
      ],
      compiler_params=pltpu.CompilerParams(use_tc_tiling_on_sc=False),
  )
  return fn(hstk, asd1, ads1, edges)


def _comb1_body(accA, accB, denA, denB, w2a_ref, w2b_ref, sl_ref, sh_ref,
                g_ref, g2_ref, b1a_ref, b1b_ref, hs2_ref, ad2_ref):
  dA = jnp.dot(denA[...], sl_ref[...],
               preferred_element_type=jnp.float32) + 1e-16
  dB = jnp.dot(denB[...], sh_ref[...],
               preferred_element_type=jnp.float32) + 1e-16
  hA = jnp.maximum(accA[...] / dA + b1a_ref[...], 0.0)
  hB = jnp.maximum(accB[...] / dB + b1b_ref[...], 0.0)
  f2 = (jnp.dot(hA, w2a_ref[...], preferred_element_type=jnp.float32)
        + jnp.dot(hB, w2b_ref[...], preferred_element_type=jnp.float32))
  hs2_ref[...] = jnp.dot(f2, g_ref[...], preferred_element_type=jnp.float32)
  ad2_ref[...] = jnp.dot(f2, g2_ref[...], preferred_element_type=jnp.float32)


def _comb1(accA, accB, denA, denB, W2a, W2b, SL, SH, G, G2, b1a, b1b,
           bn=1000):
  grid = (N // bn,)
  HW = HID // 2
  return pl.pallas_call(
      _comb1_body,
      grid=grid,
      in_specs=[
          pl.BlockSpec((bn, HW), lambda i: (i, 0)),
          pl.BlockSpec((bn, HW), lambda i: (i, 0)),
          pl.BlockSpec((bn, 16), lambda i: (i, 0)),
          pl.BlockSpec((bn, 16), lambda i: (i, 0)),
          pl.BlockSpec((HW, OUT), lambda i: (0, 0)),
          pl.BlockSpec((HW, OUT), lambda i: (0, 0)),
          pl.BlockSpec((16, HW), lambda i: (0, 0)),
          pl.BlockSpec((16, HW), lambda i: (0, 0)),
          pl.BlockSpec((OUT, 32), lambda i: (0, 0)),
          pl.BlockSpec((OUT, 16), lambda i: (0, 0)),
          pl.BlockSpec((1, HW), lambda i: (0, 0)),
          pl.BlockSpec((1, HW), lambda i: (0, 0)),
      ],
      out_specs=[
          pl.BlockSpec((bn, 32), lambda i: (i, 0)),
          pl.BlockSpec((bn, 16), lambda i: (i, 0)),
      ],
      out_shape=[
          jax.ShapeDtypeStruct((N, 32), jnp.float32),
          jax.ShapeDtypeStruct((N, 16), jnp.float32),
      ],
  )(accA, accB, denA, denB, W2a, W2b, SL, SH, G, G2, b1a, b1b)


def _edge2_body(hs2, ad2, edges, acc_out, den_out,
                idx_pk, idx_src, idx_dst, sbuf, dbuf, mbuf, wbuf, zbuf,
                s0, s1, acc_sh, den_sh):
  c = lax.axis_index("c")
  s = lax.axis_index("s")
  gw = c * NS + s

  def zrow(r, _):
    zbuf[r, :] = jnp.zeros((16,), jnp.float32)
    return 0
  lax.fori_loop(0, CHUNK, zrow, 0)
  base_rows = s * ROWS_PER_TILE
  r0 = base_rows
  for zr in ZSLICES:
    pltpu.sync_copy(zbuf.at[pl.ds(0, zr)], acc_sh.at[pl.ds(r0, zr)])
    pltpu.sync_copy(zbuf.at[pl.ds(0, zr)], den_sh.at[pl.ds(r0, zr)])
    r0 += zr
  plsc.subcore_barrier()

  zero16 = jnp.zeros((16,), jnp.int32)

  def chunk_body(i, _):
    base = pl.multiple_of(((c * NS + s) * CPW + i) * CHUNK, CHUNK)
    pltpu.async_copy(edges.at[pl.ds(base, CHUNK)], idx_pk, s0).wait()
    def unpack(q, _):
      v = idx_pk[pl.ds(q * 16, 16)]
      idx_src[pl.ds(q * 16, 16)] = v & 16383
      idx_dst[pl.ds(q * 16, 16)] = v >> 14
      return 0
    lax.fori_loop(0, CHUNK // 16, unpack, 0)
    cp0 = pltpu.async_copy(hs2.at[idx_src], sbuf, s0)
    cp1 = pltpu.async_copy(ad2.at[idx_dst], dbuf, s1)
    cp0.wait()
    cp1.wait()

    def edge_body(e, _):
      alpha = sbuf[e, pl.ds(16, 16)] + dbuf[e, :]
      w16 = jnp.exp(jnp.maximum(alpha, 0.2 * alpha))
      wb = jnp.full((16,), w16[0], jnp.float32)
      mbuf[e, :] = sbuf[e, pl.ds(0, 16)] * wb
      wbuf[e, :] = wb
      return 0
    lax.fori_loop(0, CHUNK, edge_body, 0)

    pltpu.sync_copy(mbuf, acc_sh.at[idx_dst], add=True)
    pltpu.sync_copy(wbuf, den_sh.at[idx_dst], add=True)
    return 0

  lax.fori_loop(0, CPW, chunk_body, 0)
  plsc.subcore_barrier()

  r0 = base_rows
  for zr in ZSLICES:
    pltpu.sync_copy(acc_sh.at[pl.ds(r0, zr)], acc_out.at[c, pl.ds(r0, zr)])
    pltpu.sync_copy(den_sh.at[pl.ds(r0, zr)], den_out.at[c, pl.ds(r0, zr)])
    r0 += zr


def _edge2(hs2, ad2, edges):
  mesh = plsc.VectorSubcoreMesh(core_axis_name="c", subcore_axis_name="s",
                                num_cores=NC, num_subcores=NS)
  fn = pl.kernel(
      _edge2_body,
      out_type=[
          jax.ShapeDtypeStruct((NC, NPAD, 16), jnp.float32),
          jax.ShapeDtypeStruct((NC, NPAD, 16), jnp.float32),
      ],
      mesh=mesh,
      scratch_types=[
          pltpu.VMEM((CHUNK,), jnp.int32),
          pltpu.VMEM((CHUNK,), jnp.int32),
          pltpu.VMEM((CHUNK,), jnp.int32),
          pltpu.VMEM((CHUNK, 32), jnp.float32),
          pltpu.VMEM((CHUNK, 16), jnp.float32),
          pltpu.VMEM((CHUNK, 16), jnp.float32),
          pltpu.VMEM((CHUNK, 16), jnp.float32),
          pltpu.VMEM((CHUNK, 16), jnp.float32),
          pltpu.SemaphoreType.DMA,
          pltpu.SemaphoreType.DMA,
          pltpu.VMEM_SHARED((NPAD, 16), jnp.float32),
          pltpu.VMEM_SHARED((NPAD, 16), jnp.float32),
      ],
      compiler_params=pltpu.CompilerParams(use_tc_tiling_on_sc=False),
  )
  return fn(hs2, ad2, edges)


def _final_body(a2A, a2B, d2A, d2B, b2_ref, out_ref):
  o = (a2A[...] + a2B[...]) / (d2A[...] + d2B[...] + 1e-16) + b2_ref[...]
  m = jnp.max(o, axis=1, keepdims=True)
  ex = jnp.exp(o - m)
  out_ref[...] = (o - m) - jnp.log(jnp.sum(ex, axis=1, keepdims=True))


def _final(a2A, a2B, d2A, d2B, b2, bn=1000):
  grid = (N // bn,)
  return pl.pallas_call(
      _final_body,
      grid=grid,
      in_specs=[
          pl.BlockSpec((bn, 16), lambda i: (i, 0)),
          pl.BlockSpec((bn, 16), lambda i: (i, 0)),
          pl.BlockSpec((bn, 16), lambda i: (i, 0)),
          pl.BlockSpec((bn, 16), lambda i: (i, 0)),
          pl.BlockSpec((1, 16), lambda i: (0, 0)),
      ],
      out_specs=pl.BlockSpec((bn, 16), lambda i: (i, 0)),
      out_shape=jax.ShapeDtypeStruct((N, 16), jnp.float32),
  )(a2A, a2B, d2A, d2B, b2)


@jax.jit
def kernel(x, edge_index, W1, att_src1, att_dst1, b1, W2, att_src2,
           att_dst2, b2):
  eye8 = jnp.eye(HEADS, dtype=jnp.float32)
  ab_src = (att_src1[:, :, None] * eye8[:, None, :]).reshape(HID, HEADS)
  ab_dst = (att_dst1[:, :, None] * eye8[:, None, :]).reshape(HID, HEADS)
  AB = jnp.concatenate([ab_src, ab_dst], axis=1)
  AB2 = jnp.concatenate([ab_dst, ab_src], axis=1)
  perm16 = jnp.array([4, 5, 6, 7, 0, 1, 2, 3,
                      12, 13, 14, 15, 8, 9, 10, 11])
  ATT = jnp.concatenate([AB, AB[:, perm16], AB2, AB2[:, perm16]], axis=1)

  rows = jnp.arange(16)
  cols = jnp.arange(HID // 2)
  SL = (rows[:, None] == cols[None, :] // HI).astype(jnp.float32)

  G = jnp.zeros((OUT, 32), jnp.float32)
  G = G.at[:, :OUT].set(jnp.eye(OUT, dtype=jnp.float32))
  G = G.at[:, OUT:].set(jnp.broadcast_to(att_src2[0][:, None], (OUT, 16)))
  G2 = jnp.broadcast_to(att_dst2[0][:, None], (OUT, 16)).astype(jnp.float32)

  loops = jnp.arange(N, dtype=jnp.int32)
  pad = EPAD - ETOT
  srcs = jnp.concatenate([edge_index[0], loops,
                          jnp.zeros((pad,), jnp.int32)])
  dsts = jnp.concatenate([edge_index[1], loops,
                          jnp.full((pad,), N, jnp.int32)])
  edges = srcs | (dsts << 14)

  h1a, h1b, attout = _proj1(x, W1, ATT)
  zp64 = jnp.zeros((NPAD - N, 64), jnp.float32)
  zp16 = jnp.zeros((NPAD - N, 16), jnp.float32)
  hstk = jnp.concatenate([h1a, zp64, h1b, zp64], axis=0)
  asd_stk = jnp.concatenate([attout[:, 0:16], zp16,
                             attout[:, 16:32], zp16], axis=0)
  ads_stk = jnp.concatenate([attout[:, 32:48], zp16,
                             attout[:, 48:64], zp16], axis=0)
  accs, dens = _edge1(hstk, asd_stk, ads_stk, edges)
  hs2, ad2 = _comb1(accs[0, :N], accs[1, :N], dens[0, :N], dens[1, :N],
                    W2[:HID // 2], W2[HID // 2:], SL, SL, G, G2,
                    b1[:HID // 2].reshape(1, -1), b1[HID // 2:].reshape(1, -1))
  ad2p = jnp.concatenate([ad2, zp16], axis=0)
  acc2, den2 = _edge2(hs2, ad2p, edges)
  out = _final(acc2[0, :N], acc2[1, :N], den2[0, :N], den2[1, :N],
               b2.reshape(1, OUT))
  return out


# speedup vs baseline: 48.8136x; 1.1736x over previous
import jax
import jax.numpy as jnp
from jax import lax
from jax.experimental import pallas as pl
from jax.experimental.pallas import tpu as pltpu
from jax.experimental.pallas import tpu_sc as plsc

N = 10000
E = 320000
IN_DIM = 128
HEADS = 8
HI = 16
HID = HEADS * HI
OUT = 16

NC = 2
NS = 16
NW = NC * NS

CHUNK = 128
ETOT = E + N
CPW = -(-ETOT // (NW * CHUNK))
CPW += CPW % 2
EPAD = NW * CPW * CHUNK
CPT = EPAD // (NS * CHUNK)
ROWS_PER_TILE = 626
NPAD = NS * ROWS_PER_TILE
ZSLICES = [128, 128, 128, 128, 114]
EPC = EPAD // NC


def _proj1_body(x_ref, w1_ref, att_ref, h1a_ref, h1b_ref, attout_ref):
  h = jnp.dot(x_ref[...], w1_ref[...], preferred_element_type=jnp.float32)
  h1a_ref[...] = h[:, :HID // 2]
  h1b_ref[...] = h[:, HID // 2:]
  attout_ref[...] = jnp.dot(h, att_ref[...],
                            preferred_element_type=jnp.float32)


def _proj1(x, W1, ATT, bn=1000):
  grid = (N // bn,)
  return pl.pallas_call(
      _proj1_body,
      grid=grid,
      in_specs=[
          pl.BlockSpec((bn, IN_DIM), lambda i: (i, 0)),
          pl.BlockSpec((IN_DIM, HID), lambda i: (0, 0)),
          pl.BlockSpec((HID, 64), lambda i: (0, 0)),
      ],
      out_specs=[
          pl.BlockSpec((bn, HID // 2), lambda i: (i, 0)),
          pl.BlockSpec((bn, HID // 2), lambda i: (i, 0)),
          pl.BlockSpec((bn, 64), lambda i: (i, 0)),
      ],
      out_shape=[
          jax.ShapeDtypeStruct((N, HID // 2), jnp.float32),
          jax.ShapeDtypeStruct((N, HID // 2), jnp.float32),
          jax.ShapeDtypeStruct((N, 64), jnp.float32),
      ],
  )(x, W1, ATT)


def _edge1_body(hstk, asd1, ads1, edges, acc_out, den_out,
                idx_pk,
                idx_srcA, idx_dstA, idx_dstgA, hbufA, asbufA, adbufA,
                s0A, s1A, s2A,
                idx_srcB, idx_dstB, idx_dstgB, hbufB, asbufB, adbufB,
                s0B, s1B, s2B,
                wbuf, zbuf, zbuf16, acc_sh, den_sh):
  c = lax.axis_index("c")
  s = lax.axis_index("s")
  HW = HID // 2

  def zrow(r, _):
    for j in range(HW // 16):
      zbuf[r, pl.ds(16 * j, 16)] = jnp.zeros((16,), jnp.float32)
    zbuf16[r, :] = jnp.zeros((16,), jnp.float32)
    return 0
  lax.fori_loop(0, CHUNK, zrow, 0)
  base_rows = s * ROWS_PER_TILE
  r0 = base_rows
  for zr in ZSLICES:
    pltpu.sync_copy(zbuf.at[pl.ds(0, zr)], acc_sh.at[pl.ds(r0, zr)])
    pltpu.sync_copy(zbuf16.at[pl.ds(0, zr)], den_sh.at[pl.ds(r0, zr)])
    r0 += zr
  plsc.subcore_barrier()

  coff = c * NPAD

  bufs = [
      (idx_srcA, idx_dstA, idx_dstgA, hbufA, asbufA, adbufA, s0A, s1A, s2A),
      (idx_srcB, idx_dstB, idx_dstgB, hbufB, asbufB, adbufB, s0B, s1B, s2B),
  ]

  def start(i, bset):
    idx_src, idx_dst, idx_dstg, hbuf, asbuf, adbuf, s0, s1, s2 = bset
    base = pl.multiple_of((s * CPT + i) * CHUNK, CHUNK)
    pltpu.async_copy(edges.at[pl.ds(base, CHUNK)], idx_pk, s0).wait()
    def unpack(q, _):
      v = idx_pk[pl.ds(q * 16, 16)]
      idx_src[pl.ds(q * 16, 16)] = (v & 16383) + coff
      d = v >> 14
      idx_dst[pl.ds(q * 16, 16)] = d
      idx_dstg[pl.ds(q * 16, 16)] = d + coff
      return 0
    lax.fori_loop(0, CHUNK // 16, unpack, 0, unroll=2)
    pltpu.async_copy(hstk.at[idx_src], hbuf, s0)
    pltpu.async_copy(asd1.at[idx_src], asbuf, s1)
    pltpu.async_copy(ads1.at[idx_dstg], adbuf, s2)

  def wait_gathers(bset):
    idx_src, idx_dst, idx_dstg, hbuf, asbuf, adbuf, s0, s1, s2 = bset
    pltpu.make_async_copy(hstk.at[idx_src], hbuf, s0).wait()
    pltpu.make_async_copy(asd1.at[idx_src], asbuf, s1).wait()
    pltpu.make_async_copy(ads1.at[idx_dstg], adbuf, s2).wait()

  def compute_scatter(bset):
    idx_src, idx_dst, idx_dstg, hbuf, asbuf, adbuf, s0, s1, s2 = bset

    def edge_body(e, _):
      alpha = asbuf[e, :] + adbuf[e, :]
      w16 = jnp.exp(jnp.maximum(alpha, 0.2 * alpha))
      wbuf[e, :] = w16
      for j in range(HEADS // 2):
        wb = jnp.full((16,), w16[j], jnp.float32)
        hbuf[e, pl.ds(16 * j, 16)] = hbuf[e, pl.ds(16 * j, 16)] * wb
      return 0
    lax.fori_loop(0, CHUNK, edge_body, 0, unroll=4)

    pltpu.sync_copy(hbuf, acc_sh.at[idx_dst], add=True)
    pltpu.sync_copy(wbuf, den_sh.at[idx_dst], add=True)

  start(0, bufs[0])

  def pair_body(t, _):
    for k in range(2):
      i = 2 * t + k
      wait_gathers(bufs[k])

      @pl.when(i + 1 < CPT)
      def _():
        start(i + 1, bufs[1 - k])

      compute_scatter(bufs[k])
    return 0

  lax.fori_loop(0, CPT // 2, pair_body, 0)
  plsc.subcore_barrier()

  r0 = base_rows
  for zr in ZSLICES:
    pltpu.sync_copy(acc_sh.at[pl.ds(r0, zr)], acc_out.at[c, pl.ds(r0, zr)])
    pltpu.sync_copy(den_sh.at[pl.ds(r0, zr)], den_out.at[c, pl.ds(r0, zr)])
    r0 += zr


def _edge1(hstk, asd1, ads1, edges):
  mesh = plsc.VectorSubcoreMesh(core_axis_name="c", subcore_axis_name="s",
                                num_cores=NC, num_subcores=NS)
  HW = HID // 2
  fn = pl.kernel(
      _edge1_body,
      out_type=[
          jax.ShapeDtypeStruct((NC, NPAD, HW), jnp.float32),
          jax.ShapeDtypeStruct((NC, NPAD, 16), jnp.float32),
      ],
      mesh=mesh,
      scratch_types=(
          [pltpu.VMEM((CHUNK,), jnp.int32)]
          + 2 * [
              pltpu.VMEM((CHUNK,), jnp.int32),
              pltpu.VMEM((CHUNK,), jnp.int32),
              pltpu.VMEM((CHUNK,), jnp.int32),
              pltpu.VMEM((CHUNK, HW), jnp.float32),
              pltpu.VMEM((CHUNK, 16), jnp.float32),
              pltpu.VMEM((CHUNK, 16), jnp.float32),
              pltpu.SemaphoreType.DMA,
              pltpu.SemaphoreType.DMA,
              pltpu.SemaphoreType.DMA,
          ]
          + [
              pltpu.VMEM((CHUNK, 16), jnp.float32),
              pltpu.VMEM((CHUNK, HW), jnp.float32),
              pltpu.VMEM((CHUNK, 16), jnp.float32),
              pltpu.VMEM_SHARED((NPAD, HW), jnp.float32),
              pltpu.VMEM_SHARED((NPAD, 16), jnp.float32),
          ]
      ),
      compiler_params=pltpu.CompilerParams(use_tc_tiling_on_sc=False),
  )
  return fn(hstk, asd1, ads1, edges)


def _comb1_body(accA, accB, denA, denB, w2a_ref, w2b_ref, sl_ref, sh_ref,
                g_ref, g2_ref, b1a_ref, b1b_ref, hs2_ref, ad2_ref):
  dA = jnp.dot(denA[...], sl_ref[...],
               preferred_element_type=jnp.float32) + 1e-16
  dB = jnp.dot(denB[...], sh_ref[...],
               preferred_element_type=jnp.float32) + 1e-16
  hA = jnp.maximum(accA[...] / dA + b1a_ref[...], 0.0)
  hB = jnp.maximum(accB[...] / dB + b1b_ref[...], 0.0)
  f2 = (jnp.dot(hA, w2a_ref[...], preferred_element_type=jnp.float32)
        + jnp.dot(hB, w2b_ref[...], preferred_element_type=jnp.float32))
  hs2_ref[...] = jnp.dot(f2, g_ref[...], preferred_element_type=jnp.float32)
  ad2_ref[...] = jnp.dot(f2, g2_ref[...], preferred_element_type=jnp.float32)


def _comb1(accA, accB, denA, denB, W2a, W2b, SL, SH, G, G2, b1a, b1b,
           bn=1000):
  grid = (N // bn,)
  HW = HID // 2
  return pl.pallas_call(
      _comb1_body,
      grid=grid,
      in_specs=[
          pl.BlockSpec((bn, HW), lambda i: (i, 0)),
          pl.BlockSpec((bn, HW), lambda i: (i, 0)),
          pl.BlockSpec((bn, 16), lambda i: (i, 0)),
          pl.BlockSpec((bn, 16), lambda i: (i, 0)),
          pl.BlockSpec((HW, OUT), lambda i: (0, 0)),
          pl.BlockSpec((HW, OUT), lambda i: (0, 0)),
          pl.BlockSpec((16, HW), lambda i: (0, 0)),
          pl.BlockSpec((16, HW), lambda i: (0, 0)),
          pl.BlockSpec((OUT, 32), lambda i: (0, 0)),
          pl.BlockSpec((OUT, 16), lambda i: (0, 0)),
          pl.BlockSpec((1, HW), lambda i: (0, 0)),
          pl.BlockSpec((1, HW), lambda i: (0, 0)),
      ],
      out_specs=[
          pl.BlockSpec((bn, 32), lambda i: (i, 0)),
          pl.BlockSpec((bn, 16), lambda i: (i, 0)),
      ],
      out_shape=[
          jax.ShapeDtypeStruct((N, 32), jnp.float32),
          jax.ShapeDtypeStruct((N, 16), jnp.float32),
      ],
  )(accA, accB, denA, denB, W2a, W2b, SL, SH, G, G2, b1a, b1b)


def _edge2_body(hs2, ad2, edges, acc_out, den_out,
                idx_pk,
                idx_srcA, idx_dstA, sbufA, dbufA, s0A, s1A,
                idx_srcB, idx_dstB, sbufB, dbufB, s0B, s1B,
                mbuf, wbuf, zbuf, acc_sh, den_sh):
  c = lax.axis_index("c")
  s = lax.axis_index("s")
  gw = c * NS + s

  def zrow(r, _):
    zbuf[r, :] = jnp.zeros((16,), jnp.float32)
    return 0
  lax.fori_loop(0, CHUNK, zrow, 0)
  base_rows = s * ROWS_PER_TILE
  r0 = base_rows
  for zr in ZSLICES:
    pltpu.sync_copy(zbuf.at[pl.ds(0, zr)], acc_sh.at[pl.ds(r0, zr)])
    pltpu.sync_copy(zbuf.at[pl.ds(0, zr)], den_sh.at[pl.ds(r0, zr)])
    r0 += zr
  plsc.subcore_barrier()

  bufs = [
      (idx_srcA, idx_dstA, sbufA, dbufA, s0A, s1A),
      (idx_srcB, idx_dstB, sbufB, dbufB, s0B, s1B),
  ]

  def start(i, bset):
    idx_src, idx_dst, sbuf, dbuf, s0, s1 = bset
    base = pl.multiple_of(((c * NS + s) * CPW + i) * CHUNK, CHUNK)
    pltpu.async_copy(edges.at[pl.ds(base, CHUNK)], idx_pk, s0).wait()
    def unpack(q, _):
      v = idx_pk[pl.ds(q * 16, 16)]
      idx_src[pl.ds(q * 16, 16)] = v & 16383
      idx_dst[pl.ds(q * 16, 16)] = v >> 14
      return 0
    lax.fori_loop(0, CHUNK // 16, unpack, 0, unroll=2)
    pltpu.async_copy(hs2.at[idx_src], sbuf, s0)
    pltpu.async_copy(ad2.at[idx_dst], dbuf, s1)

  def wait_gathers(bset):
    idx_src, idx_dst, sbuf, dbuf, s0, s1 = bset
    pltpu.make_async_copy(hs2.at[idx_src], sbuf, s0).wait()
    pltpu.make_async_copy(ad2.at[idx_dst], dbuf, s1).wait()

  def compute_scatter(bset):
    idx_src, idx_dst, sbuf, dbuf, s0, s1 = bset

    def edge_body(e, _):
      alpha = sbuf[e, pl.ds(16, 16)] + dbuf[e, :]
      w16 = jnp.exp(jnp.maximum(alpha, 0.2 * alpha))
      wb = jnp.full((16,), w16[0], jnp.float32)
      mbuf[e, :] = sbuf[e, pl.ds(0, 16)] * wb
      wbuf[e, :] = wb
      return 0
    lax.fori_loop(0, CHUNK, edge_body, 0, unroll=4)

    pltpu.sync_copy(mbuf, acc_sh.at[idx_dst], add=True)
    pltpu.sync_copy(wbuf, den_sh.at[idx_dst], add=True)

  start(0, bufs[0])

  def pair_body(t, _):
    for k in range(2):
      i = 2 * t + k
      wait_gathers(bufs[k])

      @pl.when(i + 1 < CPW)
      def _():
        start(i + 1, bufs[1 - k])

      compute_scatter(bufs[k])
    return 0

  lax.fori_loop(0, CPW // 2, pair_body, 0)
  plsc.subcore_barrier()

  r0 = base_rows
  for zr in ZSLICES:
    pltpu.sync_copy(acc_sh.at[pl.ds(r0, zr)], acc_out.at[c, pl.ds(r0, zr)])
    pltpu.sync_copy(den_sh.at[pl.ds(r0, zr)], den_out.at[c, pl.ds(r0, zr)])
    r0 += zr


def _edge2(hs2, ad2, edges):
  mesh = plsc.VectorSubcoreMesh(core_axis_name="c", subcore_axis_name="s",
                                num_cores=NC, num_subcores=NS)
  fn = pl.kernel(
      _edge2_body,
      out_type=[
          jax.ShapeDtypeStruct((NC, NPAD, 16), jnp.float32),
          jax.ShapeDtypeStruct((NC, NPAD, 16), jnp.float32),
      ],
      mesh=mesh,
      scratch_types=(
          [pltpu.VMEM((CHUNK,), jnp.int32)]
          + 2 * [
              pltpu.VMEM((CHUNK,), jnp.int32),
              pltpu.VMEM((CHUNK,), jnp.int32),
              pltpu.VMEM((CHUNK, 32), jnp.float32),
              pltpu.VMEM((CHUNK, 16), jnp.float32),
              pltpu.SemaphoreType.DMA,
              pltpu.SemaphoreType.DMA,
          ]
          + [
              pltpu.VMEM((CHUNK, 16), jnp.float32),
              pltpu.VMEM((CHUNK, 16), jnp.float32),
              pltpu.VMEM((CHUNK, 16), jnp.float32),
              pltpu.VMEM_SHARED((NPAD, 16), jnp.float32),
              pltpu.VMEM_SHARED((NPAD, 16), jnp.float32),
          ]
      ),
      compiler_params=pltpu.CompilerParams(use_tc_tiling_on_sc=False),
  )
  return fn(hs2, ad2, edges)


def _final_body(a2A, a2B, d2A, d2B, b2_ref, out_ref):
  o = (a2A[...] + a2B[...]) / (d2A[...] + d2B[...] + 1e-16) + b2_ref[...]
  m = jnp.max(o, axis=1, keepdims=True)
  ex = jnp.exp(o - m)
  out_ref[...] = (o - m) - jnp.log(jnp.sum(ex, axis=1, keepdims=True))


def _final(a2A, a2B, d2A, d2B, b2, bn=1000):
  grid = (N // bn,)
  return pl.pallas_call(
      _final_body,
      grid=grid,
      in_specs=[
          pl.BlockSpec((bn, 16), lambda i: (i, 0)),
          pl.BlockSpec((bn, 16), lambda i: (i, 0)),
          pl.BlockSpec((bn, 16), lambda i: (i, 0)),
          pl.BlockSpec((bn, 16), lambda i: (i, 0)),
          pl.BlockSpec((1, 16), lambda i: (0, 0)),
      ],
      out_specs=pl.BlockSpec((bn, 16), lambda i: (i, 0)),
      out_shape=jax.ShapeDtypeStruct((N, 16), jnp.float32),
  )(a2A, a2B, d2A, d2B, b2)


@jax.jit
def kernel(x, edge_index, W1, att_src1, att_dst1, b1, W2, att_src2,
           att_dst2, b2):
  eye8 = jnp.eye(HEADS, dtype=jnp.float32)
  ab_src = (att_src1[:, :, None] * eye8[:, None, :]).reshape(HID, HEADS)
  ab_dst = (att_dst1[:, :, None] * eye8[:, None, :]).reshape(HID, HEADS)
  AB = jnp.concatenate([ab_src, ab_dst], axis=1)
  AB2 = jnp.concatenate([ab_dst, ab_src], axis=1)
  perm16 = jnp.array([4, 5, 6, 7, 0, 1, 2, 3,
                      12, 13, 14, 15, 8, 9, 10, 11])
  ATT = jnp.concatenate([AB, AB[:, perm16], AB2, AB2[:, perm16]], axis=1)

  rows = jnp.arange(16)
  cols = jnp.arange(HID // 2)
  SL = (rows[:, None] == cols[None, :] // HI).astype(jnp.float32)

  G = jnp.zeros((OUT, 32), jnp.float32)
  G = G.at[:, :OUT].set(jnp.eye(OUT, dtype=jnp.float32))
  G = G.at[:, OUT:].set(jnp.broadcast_to(att_src2[0][:, None], (OUT, 16)))
  G2 = jnp.broadcast_to(att_dst2[0][:, None], (OUT, 16)).astype(jnp.float32)

  loops = jnp.arange(N, dtype=jnp.int32)
  pad = EPAD - ETOT
  srcs = jnp.concatenate([edge_index[0], loops,
                          jnp.zeros((pad,), jnp.int32)])
  dsts = jnp.concatenate([edge_index[1], loops,
                          jnp.full((pad,), N, jnp.int32)])
  edges = srcs | (dsts << 14)

  h1a, h1b, attout = _proj1(x, W1, ATT)
  zp64 = jnp.zeros((NPAD - N, 64), jnp.float32)
  zp16 = jnp.zeros((NPAD - N, 16), jnp.float32)
  hstk = jnp.concatenate([h1a, zp64, h1b, zp64], axis=0)
  asd_stk = jnp.concatenate([attout[:, 0:16], zp16,
                             attout[:, 16:32], zp16], axis=0)
  ads_stk = jnp.concatenate([attout[:, 32:48], zp16,
                             attout[:, 48:64], zp16], axis=0)
  accs, dens = _edge1(hstk, asd_stk, ads_stk, edges)
  hs2, ad2 = _comb1(accs[0, :N], accs[1, :N], dens[0, :N], dens[1, :N],
                    W2[:HID // 2], W2[HID // 2:], SL, SL, G, G2,
                    b1[:HID // 2].reshape(1, -1), b1[HID // 2:].reshape(1, -1))
  ad2p = jnp.concatenate([ad2, zp16], axis=0)
  acc2, den2 = _edge2(hs2, ad2p, edges)
  out = _final(acc2[0, :N], acc2[1, :N], den2[0, :N], den2[1, :N],
               b2.reshape(1, OUT))
  return out


# speedup vs baseline: 49.5107x; 1.0143x over previous
import jax
import jax.numpy as jnp
from jax import lax
from jax.experimental import pallas as pl
from jax.experimental.pallas import tpu as pltpu
from jax.experimental.pallas import tpu_sc as plsc

N = 10000
E = 320000
IN_DIM = 128
HEADS = 8
HI = 16
HID = HEADS * HI
OUT = 16

NC = 2
NS = 16
NW = NC * NS

CHUNK = 128
ETOT = E + N
CPW = -(-ETOT // (NW * CHUNK))
CPW += CPW % 2
EPAD = NW * CPW * CHUNK
CPT = EPAD // (NS * CHUNK)
ROWS_PER_TILE = 626
NPAD = NS * ROWS_PER_TILE
ZSLICES = [128, 128, 128, 128, 114]
EPC = EPAD // NC


TROWS = 2 * N + 8


def _proj1_body(x_ref, w1_ref, m_ref, n_ref, hstk_ref, asd_ref, ads_ref):
  x = x_ref[...]
  hstk_ref[...] = jnp.dot(x, w1_ref[0], preferred_element_type=jnp.float32)
  asd_ref[...] = jnp.dot(x, m_ref[0], preferred_element_type=jnp.float32)
  ads_ref[...] = jnp.dot(x, n_ref[0], preferred_element_type=jnp.float32)


def _proj1(x, W1, M, Nm, bn=1000):
  grid = (2, N // bn)
  return pl.pallas_call(
      _proj1_body,
      grid=grid,
      in_specs=[
          pl.BlockSpec((bn, IN_DIM), lambda f, i: (i, 0)),
          pl.BlockSpec((1, IN_DIM, HID // 2), lambda f, i: (f, 0, 0)),
          pl.BlockSpec((1, IN_DIM, 16), lambda f, i: (f, 0, 0)),
          pl.BlockSpec((1, IN_DIM, 16), lambda f, i: (f, 0, 0)),
      ],
      out_specs=[
          pl.BlockSpec((bn, HID // 2), lambda f, i: (f * (N // bn) + i, 0)),
          pl.BlockSpec((bn, 16), lambda f, i: (f * (N // bn) + i, 0)),
          pl.BlockSpec((bn, 16), lambda f, i: (f * (N // bn) + i, 0)),
      ],
      out_shape=[
          jax.ShapeDtypeStruct((TROWS, HID // 2), jnp.float32),
          jax.ShapeDtypeStruct((TROWS, 16), jnp.float32),
          jax.ShapeDtypeStruct((TROWS, 16), jnp.float32),
      ],
  )(x, W1, M, Nm)


def _edge1_body(hstk, asd1, ads1, edges, acc_out, den_out,
                idx_pk,
                idx_srcA, idx_dstA, idx_dstgA, hbufA, asbufA, adbufA,
                s0A, s1A, s2A,
                idx_srcB, idx_dstB, idx_dstgB, hbufB, asbufB, adbufB,
                s0B, s1B, s2B,
                wbuf, zbuf, zbuf16, acc_sh, den_sh):
  c = lax.axis_index("c")
  s = lax.axis_index("s")
  HW = HID // 2

  def zrow(r, _):
    for j in range(HW // 16):
      zbuf[r, pl.ds(16 * j, 16)] = jnp.zeros((16,), jnp.float32)
    zbuf16[r, :] = jnp.zeros((16,), jnp.float32)
    return 0
  lax.fori_loop(0, CHUNK, zrow, 0)
  base_rows = s * ROWS_PER_TILE
  r0 = base_rows
  for zr in ZSLICES:
    pltpu.sync_copy(zbuf.at[pl.ds(0, zr)], acc_sh.at[pl.ds(r0, zr)])
    pltpu.sync_copy(zbuf16.at[pl.ds(0, zr)], den_sh.at[pl.ds(r0, zr)])
    r0 += zr
  plsc.subcore_barrier()

  coff = c * N

  bufs = [
      (idx_srcA, idx_dstA, idx_dstgA, hbufA, asbufA, adbufA, s0A, s1A, s2A),
      (idx_srcB, idx_dstB, idx_dstgB, hbufB, asbufB, adbufB, s0B, s1B, s2B),
  ]

  def start(i, bset):
    idx_src, idx_dst, idx_dstg, hbuf, asbuf, adbuf, s0, s1, s2 = bset
    base = pl.multiple_of((s * CPT + i) * CHUNK, CHUNK)
    pltpu.async_copy(edges.at[pl.ds(base, CHUNK)], idx_pk, s0).wait()
    def unpack(q, _):
      v = idx_pk[pl.ds(q * 16, 16)]
      idx_src[pl.ds(q * 16, 16)] = (v & 16383) + coff
      d = v >> 14
      idx_dst[pl.ds(q * 16, 16)] = d
      idx_dstg[pl.ds(q * 16, 16)] = d + coff
      return 0
    lax.fori_loop(0, CHUNK // 16, unpack, 0, unroll=2)
    pltpu.async_copy(hstk.at[idx_src], hbuf, s0)
    pltpu.async_copy(asd1.at[idx_src], asbuf, s1)
    pltpu.async_copy(ads1.at[idx_dstg], adbuf, s2)

  def wait_gathers(bset):
    idx_src, idx_dst, idx_dstg, hbuf, asbuf, adbuf, s0, s1, s2 = bset
    pltpu.make_async_copy(hstk.at[idx_src], hbuf, s0).wait()
    pltpu.make_async_copy(asd1.at[idx_src], asbuf, s1).wait()
    pltpu.make_async_copy(ads1.at[idx_dstg], adbuf, s2).wait()

  def compute_scatter(bset):
    idx_src, idx_dst, idx_dstg, hbuf, asbuf, adbuf, s0, s1, s2 = bset

    def edge_body(e, _):
      alpha = asbuf[e, :] + adbuf[e, :]
      w16 = jnp.exp(jnp.maximum(alpha, 0.2 * alpha))
      wbuf[e, :] = w16
      for j in range(HEADS // 2):
        wb = jnp.full((16,), w16[j], jnp.float32)
        hbuf[e, pl.ds(16 * j, 16)] = hbuf[e, pl.ds(16 * j, 16)] * wb
      return 0
    lax.fori_loop(0, CHUNK, edge_body, 0, unroll=4)

    pltpu.sync_copy(hbuf, acc_sh.at[idx_dst], add=True)
    pltpu.sync_copy(wbuf, den_sh.at[idx_dst], add=True)

  start(0, bufs[0])

  def pair_body(t, _):
    for k in range(2):
      i = 2 * t + k
      wait_gathers(bufs[k])

      @pl.when(i + 1 < CPT)
      def _():
        start(i + 1, bufs[1 - k])

      compute_scatter(bufs[k])
    return 0

  lax.fori_loop(0, CPT // 2, pair_body, 0)
  plsc.subcore_barrier()

  r0 = base_rows
  for zr in ZSLICES:
    pltpu.sync_copy(acc_sh.at[pl.ds(r0, zr)], acc_out.at[c, pl.ds(r0, zr)])
    pltpu.sync_copy(den_sh.at[pl.ds(r0, zr)], den_out.at[c, pl.ds(r0, zr)])
    r0 += zr


def _edge1(hstk, asd1, ads1, edges):
  mesh = plsc.VectorSubcoreMesh(core_axis_name="c", subcore_axis_name="s",
                                num_cores=NC, num_subcores=NS)
  HW = HID // 2
  fn = pl.kernel(
      _edge1_body,
      out_type=[
          jax.ShapeDtypeStruct((NC, NPAD, HW), jnp.float32),
          jax.ShapeDtypeStruct((NC, NPAD, 16), jnp.float32),
      ],
      mesh=mesh,
      scratch_types=(
          [pltpu.VMEM((CHUNK,), jnp.int32)]
          + 2 * [
              pltpu.VMEM((CHUNK,), jnp.int32),
              pltpu.VMEM((CHUNK,), jnp.int32),
              pltpu.VMEM((CHUNK,), jnp.int32),
              pltpu.VMEM((CHUNK, HW), jnp.float32),
              pltpu.VMEM((CHUNK, 16), jnp.float32),
              pltpu.VMEM((CHUNK, 16), jnp.float32),
              pltpu.SemaphoreType.DMA,
              pltpu.SemaphoreType.DMA,
              pltpu.SemaphoreType.DMA,
          ]
          + [
              pltpu.VMEM((CHUNK, 16), jnp.float32),
              pltpu.VMEM((CHUNK, HW), jnp.float32),
              pltpu.VMEM((CHUNK, 16), jnp.float32),
              pltpu.VMEM_SHARED((NPAD, HW), jnp.float32),
              pltpu.VMEM_SHARED((NPAD, 16), jnp.float32),
          ]
      ),
      compiler_params=pltpu.CompilerParams(use_tc_tiling_on_sc=False),
  )
  return fn(hstk, asd1, ads1, edges)


def _comb1_body(accA, accB, denA, denB, w2a_ref, w2b_ref, sl_ref, sh_ref,
                g_ref, g2_ref, b1a_ref, b1b_ref, hs2_ref, ad2_ref):
  dA = jnp.dot(denA[...], sl_ref[...],
               preferred_element_type=jnp.float32) + 1e-16
  dB = jnp.dot(denB[...], sh_ref[...],
               preferred_element_type=jnp.float32) + 1e-16
  hA = jnp.maximum(accA[...] / dA + b1a_ref[...], 0.0)
  hB = jnp.maximum(accB[...] / dB + b1b_ref[...], 0.0)
  f2 = (jnp.dot(hA, w2a_ref[...], preferred_element_type=jnp.float32)
        + jnp.dot(hB, w2b_ref[...], preferred_element_type=jnp.float32))
  hs2_ref[...] = jnp.dot(f2, g_ref[...], preferred_element_type=jnp.float32)
  ad2_ref[...] = jnp.dot(f2, g2_ref[...], preferred_element_type=jnp.float32)


def _comb1(accA, accB, denA, denB, W2a, W2b, SL, SH, G, G2, b1a, b1b,
           bn=1000):
  grid = (N // bn,)
  HW = HID // 2
  return pl.pallas_call(
      _comb1_body,
      grid=grid,
      in_specs=[
          pl.BlockSpec((bn, HW), lambda i: (i, 0)),
          pl.BlockSpec((bn, HW), lambda i: (i, 0)),
          pl.BlockSpec((bn, 16), lambda i: (i, 0)),
          pl.BlockSpec((bn, 16), lambda i: (i, 0)),
          pl.BlockSpec((HW, OUT), lambda i: (0, 0)),
          pl.BlockSpec((HW, OUT), lambda i: (0, 0)),
          pl.BlockSpec((16, HW), lambda i: (0, 0)),
          pl.BlockSpec((16, HW), lambda i: (0, 0)),
          pl.BlockSpec((OUT, 32), lambda i: (0, 0)),
          pl.BlockSpec((OUT, 16), lambda i: (0, 0)),
          pl.BlockSpec((1, HW), lambda i: (0, 0)),
          pl.BlockSpec((1, HW), lambda i: (0, 0)),
      ],
      out_specs=[
          pl.BlockSpec((bn, 32), lambda i: (i, 0)),
          pl.BlockSpec((bn, 16), lambda i: (i, 0)),
      ],
      out_shape=[
          jax.ShapeDtypeStruct((N, 32), jnp.float32),
          jax.ShapeDtypeStruct((N, 16), jnp.float32),
      ],
  )(accA, accB, denA, denB, W2a, W2b, SL, SH, G, G2, b1a, b1b)


def _edge2_body(hs2, ad2, edges, acc_out, den_out,
                idx_pk,
                idx_srcA, idx_dstA, sbufA, dbufA, s0A, s1A,
                idx_srcB, idx_dstB, sbufB, dbufB, s0B, s1B,
                mbuf, wbuf, zbuf, acc_sh, den_sh):
  c = lax.axis_index("c")
  s = lax.axis_index("s")
  gw = c * NS + s

  def zrow(r, _):
    zbuf[r, :] = jnp.zeros((16,), jnp.float32)
    return 0
  lax.fori_loop(0, CHUNK, zrow, 0)
  base_rows = s * ROWS_PER_TILE
  r0 = base_rows
  for zr in ZSLICES:
    pltpu.sync_copy(zbuf.at[pl.ds(0, zr)], acc_sh.at[pl.ds(r0, zr)])
    pltpu.sync_copy(zbuf.at[pl.ds(0, zr)], den_sh.at[pl.ds(r0, zr)])
    r0 += zr
  plsc.subcore_barrier()

  bufs = [
      (idx_srcA, idx_dstA, sbufA, dbufA, s0A, s1A),
      (idx_srcB, idx_dstB, sbufB, dbufB, s0B, s1B),
  ]

  def start(i, bset):
    idx_src, idx_dst, sbuf, dbuf, s0, s1 = bset
    base = pl.multiple_of(((c * NS + s) * CPW + i) * CHUNK, CHUNK)
    pltpu.async_copy(edges.at[pl.ds(base, CHUNK)], idx_pk, s0).wait()
    def unpack(q, _):
      v = idx_pk[pl.ds(q * 16, 16)]
      idx_src[pl.ds(q * 16, 16)] = v & 16383
      idx_dst[pl.ds(q * 16, 16)] = v >> 14
      return 0
    lax.fori_loop(0, CHUNK // 16, unpack, 0, unroll=2)
    pltpu.async_copy(hs2.at[idx_src], sbuf, s0)
    pltpu.async_copy(ad2.at[idx_dst], dbuf, s1)

  def wait_gathers(bset):
    idx_src, idx_dst, sbuf, dbuf, s0, s1 = bset
    pltpu.make_async_copy(hs2.at[idx_src], sbuf, s0).wait()
    pltpu.make_async_copy(ad2.at[idx_dst], dbuf, s1).wait()

  def compute_scatter(bset):
    idx_src, idx_dst, sbuf, dbuf, s0, s1 = bset

    def edge_body(e, _):
      alpha = sbuf[e, pl.ds(16, 16)] + dbuf[e, :]
      w16 = jnp.exp(jnp.maximum(alpha, 0.2 * alpha))
      wb = jnp.full((16,), w16[0], jnp.float32)
      mbuf[e, :] = sbuf[e, pl.ds(0, 16)] * wb
      wbuf[e, :] = wb
      return 0
    lax.fori_loop(0, CHUNK, edge_body, 0, unroll=4)

    pltpu.sync_copy(mbuf, acc_sh.at[idx_dst], add=True)
    pltpu.sync_copy(wbuf, den_sh.at[idx_dst], add=True)

  start(0, bufs[0])

  def pair_body(t, _):
    for k in range(2):
      i = 2 * t + k
      wait_gathers(bufs[k])

      @pl.when(i + 1 < CPW)
      def _():
        start(i + 1, bufs[1 - k])

      compute_scatter(bufs[k])
    return 0

  lax.fori_loop(0, CPW // 2, pair_body, 0)
  plsc.subcore_barrier()

  r0 = base_rows
  for zr in ZSLICES:
    pltpu.sync_copy(acc_sh.at[pl.ds(r0, zr)], acc_out.at[c, pl.ds(r0, zr)])
    pltpu.sync_copy(den_sh.at[pl.ds(r0, zr)], den_out.at[c, pl.ds(r0, zr)])
    r0 += zr


def _edge2(hs2, ad2, edges):
  mesh = plsc.VectorSubcoreMesh(core_axis_name="c", subcore_axis_name="s",
                                num_cores=NC, num_subcores=NS)
  fn = pl.kernel(
      _edge2_body,
      out_type=[
          jax.ShapeDtypeStruct((NC, NPAD, 16), jnp.float32),
          jax.ShapeDtypeStruct((NC, NPAD, 16), jnp.float32),
      ],
      mesh=mesh,
      scratch_types=(
          [pltpu.VMEM((CHUNK,), jnp.int32)]
          + 2 * [
              pltpu.VMEM((CHUNK,), jnp.int32),
              pltpu.VMEM((CHUNK,), jnp.int32),
              pltpu.VMEM((CHUNK, 32), jnp.float32),
              pltpu.VMEM((CHUNK, 16), jnp.float32),
              pltpu.SemaphoreType.DMA,
              pltpu.SemaphoreType.DMA,
          ]
          + [
              pltpu.VMEM((CHUNK, 16), jnp.float32),
              pltpu.VMEM((CHUNK, 16), jnp.float32),
              pltpu.VMEM((CHUNK, 16), jnp.float32),
              pltpu.VMEM_SHARED((NPAD, 16), jnp.float32),
              pltpu.VMEM_SHARED((NPAD, 16), jnp.float32),
          ]
      ),
      compiler_params=pltpu.CompilerParams(use_tc_tiling_on_sc=False),
  )
  return fn(hs2, ad2, edges)


def _final_body(a2A, a2B, d2A, d2B, b2_ref, out_ref):
  o = (a2A[...] + a2B[...]) / (d2A[...] + d2B[...] + 1e-16) + b2_ref[...]
  m = jnp.max(o, axis=1, keepdims=True)
  ex = jnp.exp(o - m)
  out_ref[...] = (o - m) - jnp.log(jnp.sum(ex, axis=1, keepdims=True))


def _final(a2A, a2B, d2A, d2B, b2, bn=1000):
  grid = (N // bn,)
  return pl.pallas_call(
      _final_body,
      grid=grid,
      in_specs=[
          pl.BlockSpec((bn, 16), lambda i: (i, 0)),
          pl.BlockSpec((bn, 16), lambda i: (i, 0)),
          pl.BlockSpec((bn, 16), lambda i: (i, 0)),
          pl.BlockSpec((bn, 16), lambda i: (i, 0)),
          pl.BlockSpec((1, 16), lambda i: (0, 0)),
      ],
      out_specs=pl.BlockSpec((bn, 16), lambda i: (i, 0)),
      out_shape=jax.ShapeDtypeStruct((N, 16), jnp.float32),
  )(a2A, a2B, d2A, d2B, b2)


@jax.jit
def kernel(x, edge_index, W1, att_src1, att_dst1, b1, W2, att_src2,
           att_dst2, b2):
  eye8 = jnp.eye(HEADS, dtype=jnp.float32)
  ab_src = (att_src1[:, :, None] * eye8[:, None, :]).reshape(HID, HEADS)
  ab_dst = (att_dst1[:, :, None] * eye8[:, None, :]).reshape(HID, HEADS)
  AB = jnp.concatenate([ab_src, ab_dst], axis=1)
  AB2 = jnp.concatenate([ab_dst, ab_src], axis=1)
  perm16 = jnp.array([4, 5, 6, 7, 0, 1, 2, 3,
                      12, 13, 14, 15, 8, 9, 10, 11])
  ATT = jnp.concatenate([AB, AB[:, perm16], AB2, AB2[:, perm16]], axis=1)

  rows = jnp.arange(16)
  cols = jnp.arange(HID // 2)
  SL = (rows[:, None] == cols[None, :] // HI).astype(jnp.float32)

  G = jnp.zeros((OUT, 32), jnp.float32)
  G = G.at[:, :OUT].set(jnp.eye(OUT, dtype=jnp.float32))
  G = G.at[:, OUT:].set(jnp.broadcast_to(att_src2[0][:, None], (OUT, 16)))
  G2 = jnp.broadcast_to(att_dst2[0][:, None], (OUT, 16)).astype(jnp.float32)

  loops = jnp.arange(N, dtype=jnp.int32)
  pad = EPAD - ETOT
  srcs = jnp.concatenate([edge_index[0], loops,
                          jnp.zeros((pad,), jnp.int32)])
  dsts = jnp.concatenate([edge_index[1], loops,
                          jnp.full((pad,), N, jnp.int32)])
  edges = srcs | (dsts << 14)

  W1s = jnp.stack([W1[:, :HID // 2], W1[:, HID // 2:]])
  M = jnp.stack([W1 @ ATT[:, 0:16], W1 @ ATT[:, 16:32]])
  Nm = jnp.stack([W1 @ ATT[:, 32:48], W1 @ ATT[:, 48:64]])
  hstk, asd_stk, ads_stk = _proj1(x, W1s, M, Nm)
  accs, dens = _edge1(hstk, asd_stk, ads_stk, edges)
  hs2, ad2 = _comb1(accs[0, :N], accs[1, :N], dens[0, :N], dens[1, :N],
                    W2[:HID // 2], W2[HID // 2:], SL, SL, G, G2,
                    b1[:HID // 2].reshape(1, -1), b1[HID // 2:].reshape(1, -1))
  ad2p = jnp.concatenate([ad2, jnp.zeros((8, 16), jnp.float32)], axis=0)
  acc2, den2 = _edge2(hs2, ad2p, edges)
  out = _final(acc2[0, :N], acc2[1, :N], den2[0, :N], den2[1, :N],
               b2.reshape(1, OUT))
  return out


# speedup vs baseline: 49.9774x; 1.0094x over previous
import jax
import jax.numpy as jnp
from jax import lax
from jax.experimental import pallas as pl
from jax.experimental.pallas import tpu as pltpu
from jax.experimental.pallas import tpu_sc as plsc

N = 10000
E = 320000
IN_DIM = 128
HEADS = 8
HI = 16
HID = HEADS * HI
OUT = 16

NC = 2
NS = 16
NW = NC * NS

CHUNK = 128
ETOT = E + N
CPW = -(-ETOT // (NW * CHUNK))
CPW += CPW % 2
EPAD = NW * CPW * CHUNK
CPT = EPAD // (NS * CHUNK)
ROWS_PER_TILE = 626
NPAD = NS * ROWS_PER_TILE
ZSLICES = [128, 128, 128, 128, 114]
EPC = EPAD // NC


TROWS = 2 * N + 8


def _proj1_body(x_ref, w1_ref, m_ref, n_ref, hstk_ref, asd_ref, ads_ref):
  x = x_ref[...]
  hstk_ref[...] = jnp.dot(x, w1_ref[0], preferred_element_type=jnp.float32)
  asd_ref[...] = jnp.dot(x, m_ref[0], preferred_element_type=jnp.float32)
  ads_ref[...] = jnp.dot(x, n_ref[0], preferred_element_type=jnp.float32)


def _proj1(x, W1, M, Nm, bn=1000):
  grid = (2, N // bn)
  return pl.pallas_call(
      _proj1_body,
      grid=grid,
      in_specs=[
          pl.BlockSpec((bn, IN_DIM), lambda f, i: (i, 0)),
          pl.BlockSpec((1, IN_DIM, HID // 2), lambda f, i: (f, 0, 0)),
          pl.BlockSpec((1, IN_DIM, 16), lambda f, i: (f, 0, 0)),
          pl.BlockSpec((1, IN_DIM, 16), lambda f, i: (f, 0, 0)),
      ],
      out_specs=[
          pl.BlockSpec((bn, HID // 2), lambda f, i: (f * (N // bn) + i, 0)),
          pl.BlockSpec((bn, 16), lambda f, i: (f * (N // bn) + i, 0)),
          pl.BlockSpec((bn, 16), lambda f, i: (f * (N // bn) + i, 0)),
      ],
      out_shape=[
          jax.ShapeDtypeStruct((TROWS, HID // 2), jnp.float32),
          jax.ShapeDtypeStruct((TROWS, 16), jnp.float32),
          jax.ShapeDtypeStruct((TROWS, 16), jnp.float32),
      ],
  )(x, W1, M, Nm)


def _edge1_body(hstk, asd1, ads1, edges, acc_out, den_out,
                idx_pk,
                idx_srcA, idx_dstA, idx_dstgA, hbufA, asbufA, adbufA,
                wbufA, s0A, s1A, s2A, s3A, s4A,
                idx_srcB, idx_dstB, idx_dstgB, hbufB, asbufB, adbufB,
                wbufB, s0B, s1B, s2B, s3B, s4B,
                zbuf, zbuf16, acc_sh, den_sh):
  c = lax.axis_index("c")
  s = lax.axis_index("s")
  HW = HID // 2

  def zrow(r, _):
    for j in range(HW // 16):
      zbuf[r, pl.ds(16 * j, 16)] = jnp.zeros((16,), jnp.float32)
    zbuf16[r, :] = jnp.zeros((16,), jnp.float32)
    return 0
  lax.fori_loop(0, CHUNK, zrow, 0)
  base_rows = s * ROWS_PER_TILE
  r0 = base_rows
  for zr in ZSLICES:
    pltpu.sync_copy(zbuf.at[pl.ds(0, zr)], acc_sh.at[pl.ds(r0, zr)])
    pltpu.sync_copy(zbuf16.at[pl.ds(0, zr)], den_sh.at[pl.ds(r0, zr)])
    r0 += zr
  plsc.subcore_barrier()

  coff = c * N

  bufs = [
      (idx_srcA, idx_dstA, idx_dstgA, hbufA, asbufA, adbufA, wbufA,
       s0A, s1A, s2A, s3A, s4A),
      (idx_srcB, idx_dstB, idx_dstgB, hbufB, asbufB, adbufB, wbufB,
       s0B, s1B, s2B, s3B, s4B),
  ]

  def wait_scatters(bset):
    (idx_src, idx_dst, idx_dstg, hbuf, asbuf, adbuf, wbuf,
     s0, s1, s2, s3, s4) = bset
    pltpu.make_async_copy(hbuf, acc_sh.at[idx_dst], s3).wait()
    pltpu.make_async_copy(wbuf, den_sh.at[idx_dst], s4).wait()

  def start(i, bset):
    (idx_src, idx_dst, idx_dstg, hbuf, asbuf, adbuf, wbuf,
     s0, s1, s2, s3, s4) = bset

    @pl.when(i >= 2)
    def _():
      wait_scatters(bset)

    base = pl.multiple_of((s * CPT + i) * CHUNK, CHUNK)
    pltpu.async_copy(edges.at[pl.ds(base, CHUNK)], idx_pk, s0).wait()
    def unpack(q, _):
      v = idx_pk[pl.ds(q * 16, 16)]
      idx_src[pl.ds(q * 16, 16)] = (v & 16383) + coff
      d = v >> 14
      idx_dst[pl.ds(q * 16, 16)] = d
      idx_dstg[pl.ds(q * 16, 16)] = d + coff
      return 0
    lax.fori_loop(0, CHUNK // 16, unpack, 0, unroll=2)
    pltpu.async_copy(hstk.at[idx_src], hbuf, s0)
    pltpu.async_copy(asd1.at[idx_src], asbuf, s1)
    pltpu.async_copy(ads1.at[idx_dstg], adbuf, s2)

  def wait_gathers(bset):
    (idx_src, idx_dst, idx_dstg, hbuf, asbuf, adbuf, wbuf,
     s0, s1, s2, s3, s4) = bset
    pltpu.make_async_copy(hstk.at[idx_src], hbuf, s0).wait()
    pltpu.make_async_copy(asd1.at[idx_src], asbuf, s1).wait()
    pltpu.make_async_copy(ads1.at[idx_dstg], adbuf, s2).wait()

  def compute_scatter(bset):
    (idx_src, idx_dst, idx_dstg, hbuf, asbuf, adbuf, wbuf,
     s0, s1, s2, s3, s4) = bset

    def edge_body(e, _):
      alpha = asbuf[e, :] + adbuf[e, :]
      w16 = jnp.exp(jnp.maximum(alpha, 0.2 * alpha))
      wbuf[e, :] = w16
      for j in range(HEADS // 2):
        wb = jnp.full((16,), w16[j], jnp.float32)
        hbuf[e, pl.ds(16 * j, 16)] = hbuf[e, pl.ds(16 * j, 16)] * wb
      return 0
    lax.fori_loop(0, CHUNK, edge_body, 0, unroll=8)

    pltpu.async_copy(hbuf, acc_sh.at[idx_dst], s3, add=True)
    pltpu.async_copy(wbuf, den_sh.at[idx_dst], s4, add=True)

  start(0, bufs[0])

  def pair_body(t, _):
    for k in range(2):
      i = 2 * t + k
      wait_gathers(bufs[k])

      @pl.when(i + 1 < CPT)
      def _():
        start(i + 1, bufs[1 - k])

      compute_scatter(bufs[k])
    return 0

  lax.fori_loop(0, CPT // 2, pair_body, 0)
  wait_scatters(bufs[0])
  wait_scatters(bufs[1])
  plsc.subcore_barrier()

  r0 = base_rows
  for zr in ZSLICES:
    pltpu.sync_copy(acc_sh.at[pl.ds(r0, zr)], acc_out.at[c, pl.ds(r0, zr)])
    pltpu.sync_copy(den_sh.at[pl.ds(r0, zr)], den_out.at[c, pl.ds(r0, zr)])
    r0 += zr


def _edge1(hstk, asd1, ads1, edges):
  mesh = plsc.VectorSubcoreMesh(core_axis_name="c", subcore_axis_name="s",
                                num_cores=NC, num_subcores=NS)
  HW = HID // 2
  fn = pl.kernel(
      _edge1_body,
      out_type=[
          jax.ShapeDtypeStruct((NC, NPAD, HW), jnp.float32),
          jax.ShapeDtypeStruct((NC, NPAD, 16), jnp.float32),
      ],
      mesh=mesh,
      scratch_types=(
          [pltpu.VMEM((CHUNK,), jnp.int32)]
          + 2 * [
              pltpu.VMEM((CHUNK,), jnp.int32),
              pltpu.VMEM((CHUNK,), jnp.int32),
              pltpu.VMEM((CHUNK,), jnp.int32),
              pltpu.VMEM((CHUNK, HW), jnp.float32),
              pltpu.VMEM((CHUNK, 16), jnp.float32),
              pltpu.VMEM((CHUNK, 16), jnp.float32),
              pltpu.VMEM((CHUNK, 16), jnp.float32),
              pltpu.SemaphoreType.DMA,
              pltpu.SemaphoreType.DMA,
              pltpu.SemaphoreType.DMA,
              pltpu.SemaphoreType.DMA,
              pltpu.SemaphoreType.DMA,
          ]
          + [
              pltpu.VMEM((CHUNK, HW), jnp.float32),
              pltpu.VMEM((CHUNK, 16), jnp.float32),
              pltpu.VMEM_SHARED((NPAD, HW), jnp.float32),
              pltpu.VMEM_SHARED((NPAD, 16), jnp.float32),
          ]
      ),
      compiler_params=pltpu.CompilerParams(use_tc_tiling_on_sc=False),
  )
  return fn(hstk, asd1, ads1, edges)


def _comb1_body(accA, accB, denA, denB, w2a_ref, w2b_ref, sl_ref, sh_ref,
                g_ref, g2_ref, b1a_ref, b1b_ref, hs2_ref, ad2_ref):
  dA = jnp.dot(denA[...], sl_ref[...],
               preferred_element_type=jnp.float32) + 1e-16
  dB = jnp.dot(denB[...], sh_ref[...],
               preferred_element_type=jnp.float32) + 1e-16
  hA = jnp.maximum(accA[...] / dA + b1a_ref[...], 0.0)
  hB = jnp.maximum(accB[...] / dB + b1b_ref[...], 0.0)
  f2 = (jnp.dot(hA, w2a_ref[...], preferred_element_type=jnp.float32)
        + jnp.dot(hB, w2b_ref[...], preferred_element_type=jnp.float32))
  hs2_ref[...] = jnp.dot(f2, g_ref[...], preferred_element_type=jnp.float32)
  ad2_ref[...] = jnp.dot(f2, g2_ref[...], preferred_element_type=jnp.float32)


def _comb1(accA, accB, denA, denB, W2a, W2b, SL, SH, G, G2, b1a, b1b,
           bn=1000):
  grid = (N // bn,)
  HW = HID // 2
  return pl.pallas_call(
      _comb1_body,
      grid=grid,
      in_specs=[
          pl.BlockSpec((bn, HW), lambda i: (i, 0)),
          pl.BlockSpec((bn, HW), lambda i: (i, 0)),
          pl.BlockSpec((bn, 16), lambda i: (i, 0)),
          pl.BlockSpec((bn, 16), lambda i: (i, 0)),
          pl.BlockSpec((HW, OUT), lambda i: (0, 0)),
          pl.BlockSpec((HW, OUT), lambda i: (0, 0)),
          pl.BlockSpec((16, HW), lambda i: (0, 0)),
          pl.BlockSpec((16, HW), lambda i: (0, 0)),
          pl.BlockSpec((OUT, 32), lambda i: (0, 0)),
          pl.BlockSpec((OUT, 16), lambda i: (0, 0)),
          pl.BlockSpec((1, HW), lambda i: (0, 0)),
          pl.BlockSpec((1, HW), lambda i: (0, 0)),
      ],
      out_specs=[
          pl.BlockSpec((bn, 32), lambda i: (i, 0)),
          pl.BlockSpec((bn, 16), lambda i: (i, 0)),
      ],
      out_shape=[
          jax.ShapeDtypeStruct((N, 32), jnp.float32),
          jax.ShapeDtypeStruct((N, 16), jnp.float32),
      ],
  )(accA, accB, denA, denB, W2a, W2b, SL, SH, G, G2, b1a, b1b)


def _edge2_body(hs2, ad2, edges, acc_out, den_out,
                idx_pk,
                idx_srcA, idx_dstA, sbufA, dbufA, mbufA, wbufA,
                s0A, s1A, s3A, s4A,
                idx_srcB, idx_dstB, sbufB, dbufB, mbufB, wbufB,
                s0B, s1B, s3B, s4B,
                zbuf, acc_sh, den_sh):
  c = lax.axis_index("c")
  s = lax.axis_index("s")
  gw = c * NS + s

  def zrow(r, _):
    zbuf[r, :] = jnp.zeros((16,), jnp.float32)
    return 0
  lax.fori_loop(0, CHUNK, zrow, 0)
  base_rows = s * ROWS_PER_TILE
  r0 = base_rows
  for zr in ZSLICES:
    pltpu.sync_copy(zbuf.at[pl.ds(0, zr)], acc_sh.at[pl.ds(r0, zr)])
    pltpu.sync_copy(zbuf.at[pl.ds(0, zr)], den_sh.at[pl.ds(r0, zr)])
    r0 += zr
  plsc.subcore_barrier()

  bufs = [
      (idx_srcA, idx_dstA, sbufA, dbufA, mbufA, wbufA, s0A, s1A, s3A, s4A),
      (idx_srcB, idx_dstB, sbufB, dbufB, mbufB, wbufB, s0B, s1B, s3B, s4B),
  ]

  def wait_scatters(bset):
    idx_src, idx_dst, sbuf, dbuf, mbuf, wbuf, s0, s1, s3, s4 = bset
    pltpu.make_async_copy(mbuf, acc_sh.at[idx_dst], s3).wait()
    pltpu.make_async_copy(wbuf, den_sh.at[idx_dst], s4).wait()

  def start(i, bset):
    idx_src, idx_dst, sbuf, dbuf, mbuf, wbuf, s0, s1, s3, s4 = bset

    @pl.when(i >= 2)
    def _():
      wait_scatters(bset)

    base = pl.multiple_of(((c * NS + s) * CPW + i) * CHUNK, CHUNK)
    pltpu.async_copy(edges.at[pl.ds(base, CHUNK)], idx_pk, s0).wait()
    def unpack(q, _):
      v = idx_pk[pl.ds(q * 16, 16)]
      idx_src[pl.ds(q * 16, 16)] = v & 16383
      idx_dst[pl.ds(q * 16, 16)] = v >> 14
      return 0
    lax.fori_loop(0, CHUNK // 16, unpack, 0, unroll=2)
    pltpu.async_copy(hs2.at[idx_src], sbuf, s0)
    pltpu.async_copy(ad2.at[idx_dst], dbuf, s1)

  def wait_gathers(bset):
    idx_src, idx_dst, sbuf, dbuf, mbuf, wbuf, s0, s1, s3, s4 = bset
    pltpu.make_async_copy(hs2.at[idx_src], sbuf, s0).wait()
    pltpu.make_async_copy(ad2.at[idx_dst], dbuf, s1).wait()

  def compute_scatter(bset):
    idx_src, idx_dst, sbuf, dbuf, mbuf, wbuf, s0, s1, s3, s4 = bset

    def edge_body(e, _):
      alpha = sbuf[e, pl.ds(16, 16)] + dbuf[e, :]
      w16 = jnp.exp(jnp.maximum(alpha, 0.2 * alpha))
      wb = jnp.full((16,), w16[0], jnp.float32)
      mbuf[e, :] = sbuf[e, pl.ds(0, 16)] * wb
      wbuf[e, :] = wb
      return 0
    lax.fori_loop(0, CHUNK, edge_body, 0, unroll=8)

    pltpu.async_copy(mbuf, acc_sh.at[idx_dst], s3, add=True)
    pltpu.async_copy(wbuf, den_sh.at[idx_dst], s4, add=True)

  start(0, bufs[0])

  def pair_body(t, _):
    for k in range(2):
      i = 2 * t + k
      wait_gathers(bufs[k])

      @pl.when(i + 1 < CPW)
      def _():
        start(i + 1, bufs[1 - k])

      compute_scatter(bufs[k])
    return 0

  lax.fori_loop(0, CPW // 2, pair_body, 0)
  wait_scatters(bufs[0])
  wait_scatters(bufs[1])
  plsc.subcore_barrier()

  r0 = base_rows
  for zr in ZSLICES:
    pltpu.sync_copy(acc_sh.at[pl.ds(r0, zr)], acc_out.at[c, pl.ds(r0, zr)])
    pltpu.sync_copy(den_sh.at[pl.ds(r0, zr)], den_out.at[c, pl.ds(r0, zr)])
    r0 += zr


def _edge2(hs2, ad2, edges):
  mesh = plsc.VectorSubcoreMesh(core_axis_name="c", subcore_axis_name="s",
                                num_cores=NC, num_subcores=NS)
  fn = pl.kernel(
      _edge2_body,
      out_type=[
          jax.ShapeDtypeStruct((NC, NPAD, 16), jnp.float32),
          jax.ShapeDtypeStruct((NC, NPAD, 16), jnp.float32),
      ],
      mesh=mesh,
      scratch_types=(
          [pltpu.VMEM((CHUNK,), jnp.int32)]
          + 2 * [
              pltpu.VMEM((CHUNK,), jnp.int32),
              pltpu.VMEM((CHUNK,), jnp.int32),
              pltpu.VMEM((CHUNK, 32), jnp.float32),
              pltpu.VMEM((CHUNK, 16), jnp.float32),
              pltpu.VMEM((CHUNK, 16), jnp.float32),
              pltpu.VMEM((CHUNK, 16), jnp.float32),
              pltpu.SemaphoreType.DMA,
              pltpu.SemaphoreType.DMA,
              pltpu.SemaphoreType.DMA,
              pltpu.SemaphoreType.DMA,
          ]
          + [
              pltpu.VMEM((CHUNK, 16), jnp.float32),
              pltpu.VMEM_SHARED((NPAD, 16), jnp.float32),
              pltpu.VMEM_SHARED((NPAD, 16), jnp.float32),
          ]
      ),
      compiler_params=pltpu.CompilerParams(use_tc_tiling_on_sc=False),
  )
  return fn(hs2, ad2, edges)


def _final_body(a2A, a2B, d2A, d2B, b2_ref, out_ref):
  o = (a2A[...] + a2B[...]) / (d2A[...] + d2B[...] + 1e-16) + b2_ref[...]
  m = jnp.max(o, axis=1, keepdims=True)
  ex = jnp.exp(o - m)
  out_ref[...] = (o - m) - jnp.log(jnp.sum(ex, axis=1, keepdims=True))


def _final(a2A, a2B, d2A, d2B, b2, bn=1000):
  grid = (N // bn,)
  return pl.pallas_call(
      _final_body,
      grid=grid,
      in_specs=[
          pl.BlockSpec((bn, 16), lambda i: (i, 0)),
          pl.BlockSpec((bn, 16), lambda i: (i, 0)),
          pl.BlockSpec((bn, 16), lambda i: (i, 0)),
          pl.BlockSpec((bn, 16), lambda i: (i, 0)),
          pl.BlockSpec((1, 16), lambda i: (0, 0)),
      ],
      out_specs=pl.BlockSpec((bn, 16), lambda i: (i, 0)),
      out_shape=jax.ShapeDtypeStruct((N, 16), jnp.float32),
  )(a2A, a2B, d2A, d2B, b2)


@jax.jit
def kernel(x, edge_index, W1, att_src1, att_dst1, b1, W2, att_src2,
           att_dst2, b2):
  eye8 = jnp.eye(HEADS, dtype=jnp.float32)
  ab_src = (att_src1[:, :, None] * eye8[:, None, :]).reshape(HID, HEADS)
  ab_dst = (att_dst1[:, :, None] * eye8[:, None, :]).reshape(HID, HEADS)
  AB = jnp.concatenate([ab_src, ab_dst], axis=1)
  AB2 = jnp.concatenate([ab_dst, ab_src], axis=1)
  perm16 = jnp.array([4, 5, 6, 7, 0, 1, 2, 3,
                      12, 13, 14, 15, 8, 9, 10, 11])
  ATT = jnp.concatenate([AB, AB[:, perm16], AB2, AB2[:, perm16]], axis=1)

  rows = jnp.arange(16)
  cols = jnp.arange(HID // 2)
  SL = (rows[:, None] == cols[None, :] // HI).astype(jnp.float32)

  G = jnp.zeros((OUT, 32), jnp.float32)
  G = G.at[:, :OUT].set(jnp.eye(OUT, dtype=jnp.float32))
  G = G.at[:, OUT:].set(jnp.broadcast_to(att_src2[0][:, None], (OUT, 16)))
  G2 = jnp.broadcast_to(att_dst2[0][:, None], (OUT, 16)).astype(jnp.float32)

  loops = jnp.arange(N, dtype=jnp.int32)
  pad = EPAD - ETOT
  srcs = jnp.concatenate([edge_index[0], loops,
                          jnp.zeros((pad,), jnp.int32)])
  dsts = jnp.concatenate([edge_index[1], loops,
                          jnp.full((pad,), N, jnp.int32)])
  edges = srcs | (dsts << 14)

  W1s = jnp.stack([W1[:, :HID // 2], W1[:, HID // 2:]])
  M = jnp.stack([W1 @ ATT[:, 0:16], W1 @ ATT[:, 16:32]])
  Nm = jnp.stack([W1 @ ATT[:, 32:48], W1 @ ATT[:, 48:64]])
  hstk, asd_stk, ads_stk = _proj1(x, W1s, M, Nm)
  accs, dens = _edge1(hstk, asd_stk, ads_stk, edges)
  hs2, ad2 = _comb1(accs[0, :N], accs[1, :N], dens[0, :N], dens[1, :N],
                    W2[:HID // 2], W2[HID // 2:], SL, SL, G, G2,
                    b1[:HID // 2].reshape(1, -1), b1[HID // 2:].reshape(1, -1))
  ad2p = jnp.concatenate([ad2, jnp.zeros((8, 16), jnp.float32)], axis=0)
  acc2, den2 = _edge2(hs2, ad2p, edges)
  out = _final(acc2[0, :N], acc2[1, :N], den2[0, :N], den2[1, :N],
               b2.reshape(1, OUT))
  return out


# speedup vs baseline: 54.9826x; 1.1001x over previous
import jax
import jax.numpy as jnp
from jax import lax
from jax.experimental import pallas as pl
from jax.experimental.pallas import tpu as pltpu
from jax.experimental.pallas import tpu_sc as plsc

N = 10000
E = 320000
IN_DIM = 128
HEADS = 8
HI = 16
HID = HEADS * HI
OUT = 16

NC = 2
NS = 16
NW = NC * NS

CHUNK = 128
ETOT = E + N
CPW = -(-ETOT // (NW * CHUNK))
CPW += CPW % 2
EPAD = NW * CPW * CHUNK
CPT = EPAD // (NS * CHUNK)
ROWS_PER_TILE = 626
NPAD = NS * ROWS_PER_TILE
ZSLICES = [128, 128, 128, 128, 114]
EPC = EPAD // NC


TROWS = 2 * N + 8


def _proj1_body(x_ref, w1_ref, m_ref, n_ref, hstk_ref, asd_ref, ads_ref):
  x = x_ref[...]
  hstk_ref[...] = jnp.dot(x, w1_ref[0], preferred_element_type=jnp.float32)
  asd_ref[...] = jnp.dot(x, m_ref[0], preferred_element_type=jnp.float32)
  ads_ref[...] = jnp.dot(x, n_ref[0], preferred_element_type=jnp.float32)


def _proj1(x, W1, M, Nm, bn=1000):
  grid = (2, N // bn)
  return pl.pallas_call(
      _proj1_body,
      grid=grid,
      in_specs=[
          pl.BlockSpec((bn, IN_DIM), lambda f, i: (i, 0)),
          pl.BlockSpec((1, IN_DIM, HID // 2), lambda f, i: (f, 0, 0)),
          pl.BlockSpec((1, IN_DIM, 16), lambda f, i: (f, 0, 0)),
          pl.BlockSpec((1, IN_DIM, 16), lambda f, i: (f, 0, 0)),
      ],
      out_specs=[
          pl.BlockSpec((bn, HID // 2), lambda f, i: (f * (N // bn) + i, 0)),
          pl.BlockSpec((bn, 16), lambda f, i: (f * (N // bn) + i, 0)),
          pl.BlockSpec((bn, 16), lambda f, i: (f * (N // bn) + i, 0)),
      ],
      out_shape=[
          jax.ShapeDtypeStruct((TROWS, HID // 2), jnp.float32),
          jax.ShapeDtypeStruct((TROWS, 16), jnp.float32),
          jax.ShapeDtypeStruct((TROWS, 16), jnp.float32),
      ],
  )(x, W1, M, Nm)


def _edge1_body(hstk, asd1, ads1, edges, acc_out, den_out,
                epk,
                idx_srcA, idx_dstA, idx_dstgA, hbufA, asbufA, adbufA,
                wbufA, s0A, s1A, s2A, s3A, s4A,
                idx_srcB, idx_dstB, idx_dstgB, hbufB, asbufB, adbufB,
                wbufB, s0B, s1B, s2B, s3B, s4B,
                zbuf, zbuf16, acc_sh, den_sh):
  c = lax.axis_index("c")
  s = lax.axis_index("s")
  HW = HID // 2

  ecp = pltpu.async_copy(
      edges.at[pl.ds(pl.multiple_of(s * (CPT * CHUNK), CHUNK), CPT * CHUNK)],
      epk, s0A)

  def zrow(r, _):
    for j in range(HW // 16):
      zbuf[r, pl.ds(16 * j, 16)] = jnp.zeros((16,), jnp.float32)
    zbuf16[r, :] = jnp.zeros((16,), jnp.float32)
    return 0
  lax.fori_loop(0, CHUNK, zrow, 0)
  base_rows = s * ROWS_PER_TILE
  r0 = base_rows
  for zr in ZSLICES:
    pltpu.sync_copy(zbuf.at[pl.ds(0, zr)], acc_sh.at[pl.ds(r0, zr)])
    pltpu.sync_copy(zbuf16.at[pl.ds(0, zr)], den_sh.at[pl.ds(r0, zr)])
    r0 += zr
  ecp.wait()
  plsc.subcore_barrier()

  coff = c * N

  bufs = [
      (idx_srcA, idx_dstA, idx_dstgA, hbufA, asbufA, adbufA, wbufA,
       s0A, s1A, s2A, s3A, s4A),
      (idx_srcB, idx_dstB, idx_dstgB, hbufB, asbufB, adbufB, wbufB,
       s0B, s1B, s2B, s3B, s4B),
  ]

  def wait_scatters(bset):
    (idx_src, idx_dst, idx_dstg, hbuf, asbuf, adbuf, wbuf,
     s0, s1, s2, s3, s4) = bset
    pltpu.make_async_copy(hbuf, acc_sh.at[idx_dst], s3).wait()
    pltpu.make_async_copy(wbuf, den_sh.at[idx_dst], s4).wait()

  def start(i, bset):
    (idx_src, idx_dst, idx_dstg, hbuf, asbuf, adbuf, wbuf,
     s0, s1, s2, s3, s4) = bset

    @pl.when(i >= 2)
    def _():
      wait_scatters(bset)

    base = i * CHUNK
    def unpack(q, _):
      v = epk[pl.ds(base + q * 16, 16)]
      idx_src[pl.ds(q * 16, 16)] = (v & 16383) + coff
      d = v >> 14
      idx_dst[pl.ds(q * 16, 16)] = d
      idx_dstg[pl.ds(q * 16, 16)] = d + coff
      return 0
    lax.fori_loop(0, CHUNK // 16, unpack, 0, unroll=2)
    pltpu.async_copy(hstk.at[idx_src], hbuf, s0)
    pltpu.async_copy(asd1.at[idx_src], asbuf, s1)
    pltpu.async_copy(ads1.at[idx_dstg], adbuf, s2)

  def wait_gathers(bset):
    (idx_src, idx_dst, idx_dstg, hbuf, asbuf, adbuf, wbuf,
     s0, s1, s2, s3, s4) = bset
    pltpu.make_async_copy(hstk.at[idx_src], hbuf, s0).wait()
    pltpu.make_async_copy(asd1.at[idx_src], asbuf, s1).wait()
    pltpu.make_async_copy(ads1.at[idx_dstg], adbuf, s2).wait()

  def compute_scatter(bset):
    (idx_src, idx_dst, idx_dstg, hbuf, asbuf, adbuf, wbuf,
     s0, s1, s2, s3, s4) = bset

    def edge_body(e, _):
      alpha = asbuf[e, :] + adbuf[e, :]
      w16 = jnp.exp(jnp.maximum(alpha, 0.2 * alpha))
      wbuf[e, :] = w16
      for j in range(HEADS // 2):
        wb = jnp.full((16,), w16[j], jnp.float32)
        hbuf[e, pl.ds(16 * j, 16)] = hbuf[e, pl.ds(16 * j, 16)] * wb
      return 0
    lax.fori_loop(0, CHUNK, edge_body, 0, unroll=8)

    pltpu.async_copy(hbuf, acc_sh.at[idx_dst], s3, add=True)
    pltpu.async_copy(wbuf, den_sh.at[idx_dst], s4, add=True)

  start(0, bufs[0])

  def pair_body(t, _):
    for k in range(2):
      i = 2 * t + k
      wait_gathers(bufs[k])

      @pl.when(i + 1 < CPT)
      def _():
        start(i + 1, bufs[1 - k])

      compute_scatter(bufs[k])
    return 0

  lax.fori_loop(0, CPT // 2, pair_body, 0)
  wait_scatters(bufs[0])
  wait_scatters(bufs[1])
  plsc.subcore_barrier()

  r0 = base_rows
  for zr in ZSLICES:
    pltpu.sync_copy(acc_sh.at[pl.ds(r0, zr)], acc_out.at[c, pl.ds(r0, zr)])
    pltpu.sync_copy(den_sh.at[pl.ds(r0, zr)], den_out.at[c, pl.ds(r0, zr)])
    r0 += zr


def _edge1(hstk, asd1, ads1, edges):
  mesh = plsc.VectorSubcoreMesh(core_axis_name="c", subcore_axis_name="s",
                                num_cores=NC, num_subcores=NS)
  HW = HID // 2
  fn = pl.kernel(
      _edge1_body,
      out_type=[
          jax.ShapeDtypeStruct((NC, NPAD, HW), jnp.float32),
          jax.ShapeDtypeStruct((NC, NPAD, 16), jnp.float32),
      ],
      mesh=mesh,
      scratch_types=(
          [pltpu.VMEM((CPT * CHUNK,), jnp.int32)]
          + 2 * [
              pltpu.VMEM((CHUNK,), jnp.int32),
              pltpu.VMEM((CHUNK,), jnp.int32),
              pltpu.VMEM((CHUNK,), jnp.int32),
              pltpu.VMEM((CHUNK, HW), jnp.float32),
              pltpu.VMEM((CHUNK, 16), jnp.float32),
              pltpu.VMEM((CHUNK, 16), jnp.float32),
              pltpu.VMEM((CHUNK, 16), jnp.float32),
              pltpu.SemaphoreType.DMA,
              pltpu.SemaphoreType.DMA,
              pltpu.SemaphoreType.DMA,
              pltpu.SemaphoreType.DMA,
              pltpu.SemaphoreType.DMA,
          ]
          + [
              pltpu.VMEM((CHUNK, HW), jnp.float32),
              pltpu.VMEM((CHUNK, 16), jnp.float32),
              pltpu.VMEM_SHARED((NPAD, HW), jnp.float32),
              pltpu.VMEM_SHARED((NPAD, 16), jnp.float32),
          ]
      ),
      compiler_params=pltpu.CompilerParams(use_tc_tiling_on_sc=False),
  )
  return fn(hstk, asd1, ads1, edges)


def _comb1_body(accA, accB, denA, denB, w2a_ref, w2b_ref, sl_ref, sh_ref,
                g_ref, g2_ref, b1a_ref, b1b_ref, hs2_ref, ad2_ref):
  dA = jnp.dot(denA[...], sl_ref[...],
               preferred_element_type=jnp.float32) + 1e-16
  dB = jnp.dot(denB[...], sh_ref[...],
               preferred_element_type=jnp.float32) + 1e-16
  hA = jnp.maximum(accA[...] / dA + b1a_ref[...], 0.0)
  hB = jnp.maximum(accB[...] / dB + b1b_ref[...], 0.0)
  f2 = (jnp.dot(hA, w2a_ref[...], preferred_element_type=jnp.float32)
        + jnp.dot(hB, w2b_ref[...], preferred_element_type=jnp.float32))
  hs2_ref[...] = jnp.dot(f2, g_ref[...], preferred_element_type=jnp.float32)
  ad2_ref[...] = jnp.dot(f2, g2_ref[...], preferred_element_type=jnp.float32)


def _comb1(accA, accB, denA, denB, W2a, W2b, SL, SH, G, G2, b1a, b1b,
           bn=1000):
  grid = (N // bn,)
  HW = HID // 2
  return pl.pallas_call(
      _comb1_body,
      grid=grid,
      in_specs=[
          pl.BlockSpec((bn, HW), lambda i: (i, 0)),
          pl.BlockSpec((bn, HW), lambda i: (i, 0)),
          pl.BlockSpec((bn, 16), lambda i: (i, 0)),
          pl.BlockSpec((bn, 16), lambda i: (i, 0)),
          pl.BlockSpec((HW, OUT), lambda i: (0, 0)),
          pl.BlockSpec((HW, OUT), lambda i: (0, 0)),
          pl.BlockSpec((16, HW), lambda i: (0, 0)),
          pl.BlockSpec((16, HW), lambda i: (0, 0)),
          pl.BlockSpec((OUT, 32), lambda i: (0, 0)),
          pl.BlockSpec((OUT, 16), lambda i: (0, 0)),
          pl.BlockSpec((1, HW), lambda i: (0, 0)),
          pl.BlockSpec((1, HW), lambda i: (0, 0)),
      ],
      out_specs=[
          pl.BlockSpec((bn, 32), lambda i: (i, 0)),
          pl.BlockSpec((bn, 16), lambda i: (i, 0)),
      ],
      out_shape=[
          jax.ShapeDtypeStruct((N, 32), jnp.float32),
          jax.ShapeDtypeStruct((N, 16), jnp.float32),
      ],
  )(accA, accB, denA, denB, W2a, W2b, SL, SH, G, G2, b1a, b1b)


def _edge2_body(hs2, ad2, edges, acc_out, den_out,
                epk,
                idx_srcA, idx_dstA, sbufA, dbufA, mbufA, wbufA,
                s0A, s1A, s3A, s4A,
                idx_srcB, idx_dstB, sbufB, dbufB, mbufB, wbufB,
                s0B, s1B, s3B, s4B,
                zbuf, acc_sh, den_sh):
  c = lax.axis_index("c")
  s = lax.axis_index("s")
  gw = c * NS + s

  ecp = pltpu.async_copy(
      edges.at[pl.ds(pl.multiple_of((c * NS + s) * (CPW * CHUNK), CHUNK),
                     CPW * CHUNK)],
      epk, s0A)

  def zrow(r, _):
    zbuf[r, :] = jnp.zeros((16,), jnp.float32)
    return 0
  lax.fori_loop(0, CHUNK, zrow, 0)
  base_rows = s * ROWS_PER_TILE
  r0 = base_rows
  for zr in ZSLICES:
    pltpu.sync_copy(zbuf.at[pl.ds(0, zr)], acc_sh.at[pl.ds(r0, zr)])
    pltpu.sync_copy(zbuf.at[pl.ds(0, zr)], den_sh.at[pl.ds(r0, zr)])
    r0 += zr
  ecp.wait()
  plsc.subcore_barrier()

  bufs = [
      (idx_srcA, idx_dstA, sbufA, dbufA, mbufA, wbufA, s0A, s1A, s3A, s4A),
      (idx_srcB, idx_dstB, sbufB, dbufB, mbufB, wbufB, s0B, s1B, s3B, s4B),
  ]

  def wait_scatters(bset):
    idx_src, idx_dst, sbuf, dbuf, mbuf, wbuf, s0, s1, s3, s4 = bset
    pltpu.make_async_copy(mbuf, acc_sh.at[idx_dst], s3).wait()
    pltpu.make_async_copy(wbuf, den_sh.at[idx_dst], s4).wait()

  def start(i, bset):
    idx_src, idx_dst, sbuf, dbuf, mbuf, wbuf, s0, s1, s3, s4 = bset

    @pl.when(i >= 2)
    def _():
      wait_scatters(bset)

    base = i * CHUNK
    def unpack(q, _):
      v = epk[pl.ds(base + q * 16, 16)]
      idx_src[pl.ds(q * 16, 16)] = v & 16383
      idx_dst[pl.ds(q * 16, 16)] = v >> 14
      return 0
    lax.fori_loop(0, CHUNK // 16, unpack, 0, unroll=2)
    pltpu.async_copy(hs2.at[idx_src], sbuf, s0)
    pltpu.async_copy(ad2.at[idx_dst], dbuf, s1)

  def wait_gathers(bset):
    idx_src, idx_dst, sbuf, dbuf, mbuf, wbuf, s0, s1, s3, s4 = bset
    pltpu.make_async_copy(hs2.at[idx_src], sbuf, s0).wait()
    pltpu.make_async_copy(ad2.at[idx_dst], dbuf, s1).wait()

  def compute_scatter(bset):
    idx_src, idx_dst, sbuf, dbuf, mbuf, wbuf, s0, s1, s3, s4 = bset

    def edge_body(e, _):
      alpha = sbuf[e, pl.ds(16, 16)] + dbuf[e, :]
      w16 = jnp.exp(jnp.maximum(alpha, 0.2 * alpha))
      wb = jnp.full((16,), w16[0], jnp.float32)
      mbuf[e, :] = sbuf[e, pl.ds(0, 16)] * wb
      wbuf[e, :] = wb
      return 0
    lax.fori_loop(0, CHUNK, edge_body, 0, unroll=8)

    pltpu.async_copy(mbuf, acc_sh.at[idx_dst], s3, add=True)
    pltpu.async_copy(wbuf, den_sh.at[idx_dst], s4, add=True)

  start(0, bufs[0])

  def pair_body(t, _):
    for k in range(2):
      i = 2 * t + k
      wait_gathers(bufs[k])

      @pl.when(i + 1 < CPW)
      def _():
        start(i + 1, bufs[1 - k])

      compute_scatter(bufs[k])
    return 0

  lax.fori_loop(0, CPW // 2, pair_body, 0)
  wait_scatters(bufs[0])
  wait_scatters(bufs[1])
  plsc.subcore_barrier()

  r0 = base_rows
  for zr in ZSLICES:
    pltpu.sync_copy(acc_sh.at[pl.ds(r0, zr)], acc_out.at[c, pl.ds(r0, zr)])
    pltpu.sync_copy(den_sh.at[pl.ds(r0, zr)], den_out.at[c, pl.ds(r0, zr)])
    r0 += zr


def _edge2(hs2, ad2, edges):
  mesh = plsc.VectorSubcoreMesh(core_axis_name="c", subcore_axis_name="s",
                                num_cores=NC, num_subcores=NS)
  fn = pl.kernel(
      _edge2_body,
      out_type=[
          jax.ShapeDtypeStruct((NC, NPAD, 16), jnp.float32),
          jax.ShapeDtypeStruct((NC, NPAD, 16), jnp.float32),
      ],
      mesh=mesh,
      scratch_types=(
          [pltpu.VMEM((CPW * CHUNK,), jnp.int32)]
          + 2 * [
              pltpu.VMEM((CHUNK,), jnp.int32),
              pltpu.VMEM((CHUNK,), jnp.int32),
              pltpu.VMEM((CHUNK, 32), jnp.float32),
              pltpu.VMEM((CHUNK, 16), jnp.float32),
              pltpu.VMEM((CHUNK, 16), jnp.float32),
              pltpu.VMEM((CHUNK, 16), jnp.float32),
              pltpu.SemaphoreType.DMA,
              pltpu.SemaphoreType.DMA,
              pltpu.SemaphoreType.DMA,
              pltpu.SemaphoreType.DMA,
          ]
          + [
              pltpu.VMEM((CHUNK, 16), jnp.float32),
              pltpu.VMEM_SHARED((NPAD, 16), jnp.float32),
              pltpu.VMEM_SHARED((NPAD, 16), jnp.float32),
          ]
      ),
      compiler_params=pltpu.CompilerParams(use_tc_tiling_on_sc=False),
  )
  return fn(hs2, ad2, edges)


def _final_body(a2A, a2B, d2A, d2B, b2_ref, out_ref):
  o = (a2A[...] + a2B[...]) / (d2A[...] + d2B[...] + 1e-16) + b2_ref[...]
  m = jnp.max(o, axis=1, keepdims=True)
  ex = jnp.exp(o - m)
  out_ref[...] = (o - m) - jnp.log(jnp.sum(ex, axis=1, keepdims=True))


def _final(a2A, a2B, d2A, d2B, b2, bn=1000):
  grid = (N // bn,)
  return pl.pallas_call(
      _final_body,
      grid=grid,
      in_specs=[
          pl.BlockSpec((bn, 16), lambda i: (i, 0)),
          pl.BlockSpec((bn, 16), lambda i: (i, 0)),
          pl.BlockSpec((bn, 16), lambda i: (i, 0)),
          pl.BlockSpec((bn, 16), lambda i: (i, 0)),
          pl.BlockSpec((1, 16), lambda i: (0, 0)),
      ],
      out_specs=pl.BlockSpec((bn, 16), lambda i: (i, 0)),
      out_shape=jax.ShapeDtypeStruct((N, 16), jnp.float32),
  )(a2A, a2B, d2A, d2B, b2)


@jax.jit
def kernel(x, edge_index, W1, att_src1, att_dst1, b1, W2, att_src2,
           att_dst2, b2):
  eye8 = jnp.eye(HEADS, dtype=jnp.float32)
  ab_src = (att_src1[:, :, None] * eye8[:, None, :]).reshape(HID, HEADS)
  ab_dst = (att_dst1[:, :, None] * eye8[:, None, :]).reshape(HID, HEADS)
  AB = jnp.concatenate([ab_src, ab_dst], axis=1)
  AB2 = jnp.concatenate([ab_dst, ab_src], axis=1)
  perm16 = jnp.array([4, 5, 6, 7, 0, 1, 2, 3,
                      12, 13, 14, 15, 8, 9, 10, 11])
  ATT = jnp.concatenate([AB, AB[:, perm16], AB2, AB2[:, perm16]], axis=1)

  rows = jnp.arange(16)
  cols = jnp.arange(HID // 2)
  SL = (rows[:, None] == cols[None, :] // HI).astype(jnp.float32)

  G = jnp.zeros((OUT, 32), jnp.float32)
  G = G.at[:, :OUT].set(jnp.eye(OUT, dtype=jnp.float32))
  G = G.at[:, OUT:].set(jnp.broadcast_to(att_src2[0][:, None], (OUT, 16)))
  G2 = jnp.broadcast_to(att_dst2[0][:, None], (OUT, 16)).astype(jnp.float32)

  loops = jnp.arange(N, dtype=jnp.int32)
  pad = EPAD - ETOT
  srcs = jnp.concatenate([edge_index[0], loops,
                          jnp.zeros((pad,), jnp.int32)])
  dsts = jnp.concatenate([edge_index[1], loops,
                          jnp.full((pad,), N, jnp.int32)])
  edges = srcs | (dsts << 14)

  W1s = jnp.stack([W1[:, :HID // 2], W1[:, HID // 2:]])
  M = jnp.stack([W1 @ ATT[:, 0:16], W1 @ ATT[:, 16:32]])
  Nm = jnp.stack([W1 @ ATT[:, 32:48], W1 @ ATT[:, 48:64]])
  hstk, asd_stk, ads_stk = _proj1(x, W1s, M, Nm)
  accs, dens = _edge1(hstk, asd_stk, ads_stk, edges)
  hs2, ad2 = _comb1(accs[0, :N], accs[1, :N], dens[0, :N], dens[1, :N],
                    W2[:HID // 2], W2[HID // 2:], SL, SL, G, G2,
                    b1[:HID // 2].reshape(1, -1), b1[HID // 2:].reshape(1, -1))
  ad2p = jnp.concatenate([ad2, jnp.zeros((8, 16), jnp.float32)], axis=0)
  acc2, den2 = _edge2(hs2, ad2p, edges)
  out = _final(acc2[0, :N], acc2[1, :N], den2[0, :N], den2[1, :N],
               b2.reshape(1, OUT))
  return out


# speedup vs baseline: 76.3467x; 1.3886x over previous
import jax
import jax.numpy as jnp
from jax import lax
from jax.experimental import pallas as pl
from jax.experimental.pallas import tpu as pltpu
from jax.experimental.pallas import tpu_sc as plsc

N = 10000
E = 320000
IN_DIM = 128
HEADS = 8
HI = 16
HID = HEADS * HI
OUT = 16

NC = 2
NS = 16
NW = NC * NS

CHUNK = 128
ETOT = E + N
CPW = -(-ETOT // (NW * CHUNK))
CPW += CPW % 2
EPAD = NW * CPW * CHUNK
CPT = EPAD // (NS * CHUNK)
ROWS_PER_TILE = 626
NPAD = NS * ROWS_PER_TILE
ZSLICES = [128, 128, 128, 128, 114]


TROWS = 2 * N + 8


def _proj1_body(x_ref, w1_ref, m_ref, n_ref, hstk_ref, asd_ref, ads_ref):
  x = x_ref[...]
  hstk_ref[...] = jnp.dot(x, w1_ref[0], preferred_element_type=jnp.float32)
  asd_ref[...] = jnp.dot(x, m_ref[0], preferred_element_type=jnp.float32)
  ads_ref[...] = jnp.dot(x, n_ref[0], preferred_element_type=jnp.float32)


def _proj1(x, W1, M, Nm, bn=1000):
  grid = (2, N // bn)
  return pl.pallas_call(
      _proj1_body,
      grid=grid,
      in_specs=[
          pl.BlockSpec((bn, IN_DIM), lambda f, i: (i, 0)),
          pl.BlockSpec((1, IN_DIM, HID // 2), lambda f, i: (f, 0, 0)),
          pl.BlockSpec((1, IN_DIM, 16), lambda f, i: (f, 0, 0)),
          pl.BlockSpec((1, IN_DIM, 16), lambda f, i: (f, 0, 0)),
      ],
      out_specs=[
          pl.BlockSpec((bn, HID // 2), lambda f, i: (f * (N // bn) + i, 0)),
          pl.BlockSpec((bn, 16), lambda f, i: (f * (N // bn) + i, 0)),
          pl.BlockSpec((bn, 16), lambda f, i: (f * (N // bn) + i, 0)),
      ],
      out_shape=[
          jax.ShapeDtypeStruct((TROWS, HID // 2), jnp.float32),
          jax.ShapeDtypeStruct((TROWS, 16), jnp.float32),
          jax.ShapeDtypeStruct((TROWS, 16), jnp.float32),
      ],
  )(x, W1, M, Nm)


def _edge1_body(hstk, asd1, ads1, edges, acc_out, den_out,
                epk,
                idx_srcA, idx_dstA, idx_dstgA, hbufA, asbufA, adbufA,
                wbufA, s0A, s1A, s2A, s3A, s4A,
                idx_srcB, idx_dstB, idx_dstgB, hbufB, asbufB, adbufB,
                wbufB, s0B, s1B, s2B, s3B, s4B,
                zbuf, zbuf16, acc_sh, den_sh):
  c = lax.axis_index("c")
  s = lax.axis_index("s")
  HW = HID // 2

  ecp = pltpu.async_copy(
      edges.at[pl.ds(pl.multiple_of(s * (CPT * CHUNK), CHUNK), CPT * CHUNK)],
      epk, s0A)

  def zrow(r, _):
    for j in range(HW // 16):
      zbuf[r, pl.ds(16 * j, 16)] = jnp.zeros((16,), jnp.float32)
    zbuf16[r, :] = jnp.zeros((16,), jnp.float32)
    return 0
  lax.fori_loop(0, CHUNK, zrow, 0)
  base_rows = s * ROWS_PER_TILE
  r0 = base_rows
  for zr in ZSLICES:
    pltpu.sync_copy(zbuf.at[pl.ds(0, zr)], acc_sh.at[pl.ds(r0, zr)])
    pltpu.sync_copy(zbuf16.at[pl.ds(0, zr)], den_sh.at[pl.ds(r0, zr)])
    r0 += zr
  ecp.wait()
  plsc.subcore_barrier()

  coff = c * N

  bufs = [
      (idx_srcA, idx_dstA, idx_dstgA, hbufA, asbufA, adbufA, wbufA,
       s0A, s1A, s2A, s3A, s4A),
      (idx_srcB, idx_dstB, idx_dstgB, hbufB, asbufB, adbufB, wbufB,
       s0B, s1B, s2B, s3B, s4B),
  ]

  def wait_scatters(bset):
    (idx_src, idx_dst, idx_dstg, hbuf, asbuf, adbuf, wbuf,
     s0, s1, s2, s3, s4) = bset
    pltpu.make_async_copy(hbuf, acc_sh.at[idx_dst], s3).wait()
    pltpu.make_async_copy(wbuf, den_sh.at[idx_dst], s4).wait()

  def start(i, bset):
    (idx_src, idx_dst, idx_dstg, hbuf, asbuf, adbuf, wbuf,
     s0, s1, s2, s3, s4) = bset

    @pl.when(i >= 2)
    def _():
      wait_scatters(bset)

    base = i * CHUNK
    def unpack(q, _):
      v = epk[pl.ds(base + q * 16, 16)]
      idx_src[pl.ds(q * 16, 16)] = (v & 16383) + coff
      d = v >> 14
      idx_dst[pl.ds(q * 16, 16)] = d
      idx_dstg[pl.ds(q * 16, 16)] = d + coff
      return 0
    lax.fori_loop(0, CHUNK // 16, unpack, 0, unroll=2)
    pltpu.async_copy(hstk.at[idx_src], hbuf, s0)
    pltpu.async_copy(asd1.at[idx_src], asbuf, s1)
    pltpu.async_copy(ads1.at[idx_dstg], adbuf, s2)

  def wait_gathers(bset):
    (idx_src, idx_dst, idx_dstg, hbuf, asbuf, adbuf, wbuf,
     s0, s1, s2, s3, s4) = bset
    pltpu.make_async_copy(hstk.at[idx_src], hbuf, s0).wait()
    pltpu.make_async_copy(asd1.at[idx_src], asbuf, s1).wait()
    pltpu.make_async_copy(ads1.at[idx_dstg], adbuf, s2).wait()

  def compute_scatter(bset):
    (idx_src, idx_dst, idx_dstg, hbuf, asbuf, adbuf, wbuf,
     s0, s1, s2, s3, s4) = bset

    @plsc.parallel_loop(0, CHUNK, step=1, unroll=8)
    def edge_body(e):
      alpha = asbuf[e, :] + adbuf[e, :]
      w16 = jnp.exp(jnp.maximum(alpha, 0.2 * alpha))
      wbuf[e, :] = w16
      for j in range(HEADS // 2):
        wb = jnp.full((16,), w16[j], jnp.float32)
        hbuf[e, pl.ds(16 * j, 16)] = hbuf[e, pl.ds(16 * j, 16)] * wb

    pltpu.async_copy(hbuf, acc_sh.at[idx_dst], s3, add=True)
    pltpu.async_copy(wbuf, den_sh.at[idx_dst], s4, add=True)

  start(0, bufs[0])

  def pair_body(t, _):
    for k in range(2):
      i = 2 * t + k
      wait_gathers(bufs[k])

      @pl.when(i + 1 < CPT)
      def _():
        start(i + 1, bufs[1 - k])

      compute_scatter(bufs[k])
    return 0

  lax.fori_loop(0, CPT // 2, pair_body, 0)
  wait_scatters(bufs[0])
  wait_scatters(bufs[1])
  plsc.subcore_barrier()

  r0 = base_rows
  for zr in ZSLICES:
    pltpu.sync_copy(acc_sh.at[pl.ds(r0, zr)], acc_out.at[c, pl.ds(r0, zr)])
    pltpu.sync_copy(den_sh.at[pl.ds(r0, zr)], den_out.at[c, pl.ds(r0, zr)])
    r0 += zr


def _edge1(hstk, asd1, ads1, edges):
  mesh = plsc.VectorSubcoreMesh(core_axis_name="c", subcore_axis_name="s",
                                num_cores=NC, num_subcores=NS)
  HW = HID // 2
  fn = pl.kernel(
      _edge1_body,
      out_type=[
          jax.ShapeDtypeStruct((NC, NPAD, HW), jnp.float32),
          jax.ShapeDtypeStruct((NC, NPAD, 16), jnp.float32),
      ],
      mesh=mesh,
      scratch_types=(
          [pltpu.VMEM((CPT * CHUNK,), jnp.int32)]
          + 2 * [
              pltpu.VMEM((CHUNK,), jnp.int32),
              pltpu.VMEM((CHUNK,), jnp.int32),
              pltpu.VMEM((CHUNK,), jnp.int32),
              pltpu.VMEM((CHUNK, HW), jnp.float32),
              pltpu.VMEM((CHUNK, 16), jnp.float32),
              pltpu.VMEM((CHUNK, 16), jnp.float32),
              pltpu.VMEM((CHUNK, 16), jnp.float32),
              pltpu.SemaphoreType.DMA,
              pltpu.SemaphoreType.DMA,
              pltpu.SemaphoreType.DMA,
              pltpu.SemaphoreType.DMA,
              pltpu.SemaphoreType.DMA,
          ]
          + [
              pltpu.VMEM((CHUNK, HW), jnp.float32),
              pltpu.VMEM((CHUNK, 16), jnp.float32),
              pltpu.VMEM_SHARED((NPAD, HW), jnp.float32),
              pltpu.VMEM_SHARED((NPAD, 16), jnp.float32),
          ]
      ),
      compiler_params=pltpu.CompilerParams(use_tc_tiling_on_sc=False),
  )
  return fn(hstk, asd1, ads1, edges)


def _comb1_body(accA, accB, denA, denB, w2a_ref, w2b_ref, sl_ref, sh_ref,
                g_ref, g2_ref, b1a_ref, b1b_ref, hs2_ref, ad2_ref):
  dA = jnp.dot(denA[...], sl_ref[...],
               preferred_element_type=jnp.float32) + 1e-16
  dB = jnp.dot(denB[...], sh_ref[...],
               preferred_element_type=jnp.float32) + 1e-16
  hA = jnp.maximum(accA[...] / dA + b1a_ref[...], 0.0)
  hB = jnp.maximum(accB[...] / dB + b1b_ref[...], 0.0)
  f2 = (jnp.dot(hA, w2a_ref[...], preferred_element_type=jnp.float32)
        + jnp.dot(hB, w2b_ref[...], preferred_element_type=jnp.float32))
  hs2_ref[...] = jnp.dot(f2, g_ref[...], preferred_element_type=jnp.float32)
  ad2_ref[...] = jnp.dot(f2, g2_ref[...], preferred_element_type=jnp.float32)


def _comb1(accA, accB, denA, denB, W2a, W2b, SL, SH, G, G2, b1a, b1b,
           bn=1000):
  grid = (N // bn,)
  HW = HID // 2
  return pl.pallas_call(
      _comb1_body,
      grid=grid,
      in_specs=[
          pl.BlockSpec((bn, HW), lambda i: (i, 0)),
          pl.BlockSpec((bn, HW), lambda i: (i, 0)),
          pl.BlockSpec((bn, 16), lambda i: (i, 0)),
          pl.BlockSpec((bn, 16), lambda i: (i, 0)),
          pl.BlockSpec((HW, OUT), lambda i: (0, 0)),
          pl.BlockSpec((HW, OUT), lambda i: (0, 0)),
          pl.BlockSpec((16, HW), lambda i: (0, 0)),
          pl.BlockSpec((16, HW), lambda i: (0, 0)),
          pl.BlockSpec((OUT, 32), lambda i: (0, 0)),
          pl.BlockSpec((OUT, 16), lambda i: (0, 0)),
          pl.BlockSpec((1, HW), lambda i: (0, 0)),
          pl.BlockSpec((1, HW), lambda i: (0, 0)),
      ],
      out_specs=[
          pl.BlockSpec((bn, 32), lambda i: (i, 0)),
          pl.BlockSpec((bn, 16), lambda i: (i, 0)),
      ],
      out_shape=[
          jax.ShapeDtypeStruct((N, 32), jnp.float32),
          jax.ShapeDtypeStruct((N, 16), jnp.float32),
      ],
  )(accA, accB, denA, denB, W2a, W2b, SL, SH, G, G2, b1a, b1b)


def _edge2_body(hs2, ad2, edges, acc_out, den_out,
                epk,
                idx_srcA, idx_dstA, sbufA, dbufA, mbufA, wbufA,
                s0A, s1A, s3A, s4A,
                idx_srcB, idx_dstB, sbufB, dbufB, mbufB, wbufB,
                s0B, s1B, s3B, s4B,
                zbuf, acc_sh, den_sh):
  c = lax.axis_index("c")
  s = lax.axis_index("s")
  gw = c * NS + s

  ecp = pltpu.async_copy(
      edges.at[pl.ds(pl.multiple_of((c * NS + s) * (CPW * CHUNK), CHUNK),
                     CPW * CHUNK)],
      epk, s0A)

  def zrow(r, _):
    zbuf[r, :] = jnp.zeros((16,), jnp.float32)
    return 0
  lax.fori_loop(0, CHUNK, zrow, 0)
  base_rows = s * ROWS_PER_TILE
  r0 = base_rows
  for zr in ZSLICES:
    pltpu.sync_copy(zbuf.at[pl.ds(0, zr)], acc_sh.at[pl.ds(r0, zr)])
    pltpu.sync_copy(zbuf.at[pl.ds(0, zr)], den_sh.at[pl.ds(r0, zr)])
    r0 += zr
  ecp.wait()
  plsc.subcore_barrier()

  bufs = [
      (idx_srcA, idx_dstA, sbufA, dbufA, mbufA, wbufA, s0A, s1A, s3A, s4A),
      (idx_srcB, idx_dstB, sbufB, dbufB, mbufB, wbufB, s0B, s1B, s3B, s4B),
  ]

  def wait_scatters(bset):
    idx_src, idx_dst, sbuf, dbuf, mbuf, wbuf, s0, s1, s3, s4 = bset
    pltpu.make_async_copy(mbuf, acc_sh.at[idx_dst], s3).wait()
    pltpu.make_async_copy(wbuf, den_sh.at[idx_dst], s4).wait()

  def start(i, bset):
    idx_src, idx_dst, sbuf, dbuf, mbuf, wbuf, s0, s1, s3, s4 = bset

    @pl.when(i >= 2)
    def _():
      wait_scatters(bset)

    base = i * CHUNK
    def unpack(q, _):
      v = epk[pl.ds(base + q * 16, 16)]
      idx_src[pl.ds(q * 16, 16)] = v & 16383
      idx_dst[pl.ds(q * 16, 16)] = v >> 14
      return 0
    lax.fori_loop(0, CHUNK // 16, unpack, 0, unroll=2)
    pltpu.async_copy(hs2.at[idx_src], sbuf, s0)
    pltpu.async_copy(ad2.at[idx_dst], dbuf, s1)

  def wait_gathers(bset):
    idx_src, idx_dst, sbuf, dbuf, mbuf, wbuf, s0, s1, s3, s4 = bset
    pltpu.make_async_copy(hs2.at[idx_src], sbuf, s0).wait()
    pltpu.make_async_copy(ad2.at[idx_dst], dbuf, s1).wait()

  def compute_scatter(bset):
    idx_src, idx_dst, sbuf, dbuf, mbuf, wbuf, s0, s1, s3, s4 = bset

    @plsc.parallel_loop(0, CHUNK, step=1, unroll=8)
    def edge_body(e):
      alpha = sbuf[e, pl.ds(16, 16)] + dbuf[e, :]
      w16 = jnp.exp(jnp.maximum(alpha, 0.2 * alpha))
      wb = jnp.full((16,), w16[0], jnp.float32)
      mbuf[e, :] = sbuf[e, pl.ds(0, 16)] * wb
      wbuf[e, :] = wb

    pltpu.async_copy(mbuf, acc_sh.at[idx_dst], s3, add=True)
    pltpu.async_copy(wbuf, den_sh.at[idx_dst], s4, add=True)

  start(0, bufs[0])

  def pair_body(t, _):
    for k in range(2):
      i = 2 * t + k
      wait_gathers(bufs[k])

      @pl.when(i + 1 < CPW)
      def _():
        start(i + 1, bufs[1 - k])

      compute_scatter(bufs[k])
    return 0

  lax.fori_loop(0, CPW // 2, pair_body, 0)
  wait_scatters(bufs[0])
  wait_scatters(bufs[1])
  plsc.subcore_barrier()

  r0 = base_rows
  for zr in ZSLICES:
    pltpu.sync_copy(acc_sh.at[pl.ds(r0, zr)], acc_out.at[c, pl.ds(r0, zr)])
    pltpu.sync_copy(den_sh.at[pl.ds(r0, zr)], den_out.at[c, pl.ds(r0, zr)])
    r0 += zr


def _edge2(hs2, ad2, edges):
  mesh = plsc.VectorSubcoreMesh(core_axis_name="c", subcore_axis_name="s",
                                num_cores=NC, num_subcores=NS)
  fn = pl.kernel(
      _edge2_body,
      out_type=[
          jax.ShapeDtypeStruct((NC, NPAD, 16), jnp.float32),
          jax.ShapeDtypeStruct((NC, NPAD, 16), jnp.float32),
      ],
      mesh=mesh,
      scratch_types=(
          [pltpu.VMEM((CPW * CHUNK,), jnp.int32)]
          + 2 * [
              pltpu.VMEM((CHUNK,), jnp.int32),
              pltpu.VMEM((CHUNK,), jnp.int32),
              pltpu.VMEM((CHUNK, 32), jnp.float32),
              pltpu.VMEM((CHUNK, 16), jnp.float32),
              pltpu.VMEM((CHUNK, 16), jnp.float32),
              pltpu.VMEM((CHUNK, 16), jnp.float32),
              pltpu.SemaphoreType.DMA,
              pltpu.SemaphoreType.DMA,
              pltpu.SemaphoreType.DMA,
              pltpu.SemaphoreType.DMA,
          ]
          + [
              pltpu.VMEM((CHUNK, 16), jnp.float32),
              pltpu.VMEM_SHARED((NPAD, 16), jnp.float32),
              pltpu.VMEM_SHARED((NPAD, 16), jnp.float32),
          ]
      ),
      compiler_params=pltpu.CompilerParams(use_tc_tiling_on_sc=False),
  )
  return fn(hs2, ad2, edges)


def _final_body(a2A, a2B, d2A, d2B, b2_ref, out_ref):
  o = (a2A[...] + a2B[...]) / (d2A[...] + d2B[...] + 1e-16) + b2_ref[...]
  m = jnp.max(o, axis=1, keepdims=True)
  ex = jnp.exp(o - m)
  out_ref[...] = (o - m) - jnp.log(jnp.sum(ex, axis=1, keepdims=True))


def _final(a2A, a2B, d2A, d2B, b2, bn=1000):
  grid = (N // bn,)
  return pl.pallas_call(
      _final_body,
      grid=grid,
      in_specs=[
          pl.BlockSpec((bn, 16), lambda i: (i, 0)),
          pl.BlockSpec((bn, 16), lambda i: (i, 0)),
          pl.BlockSpec((bn, 16), lambda i: (i, 0)),
          pl.BlockSpec((bn, 16), lambda i: (i, 0)),
          pl.BlockSpec((1, 16), lambda i: (0, 0)),
      ],
      out_specs=pl.BlockSpec((bn, 16), lambda i: (i, 0)),
      out_shape=jax.ShapeDtypeStruct((N, 16), jnp.float32),
  )(a2A, a2B, d2A, d2B, b2)


@jax.jit
def kernel(x, edge_index, W1, att_src1, att_dst1, b1, W2, att_src2,
           att_dst2, b2):
  eye8 = jnp.eye(HEADS, dtype=jnp.float32)
  ab_src = (att_src1[:, :, None] * eye8[:, None, :]).reshape(HID, HEADS)
  ab_dst = (att_dst1[:, :, None] * eye8[:, None, :]).reshape(HID, HEADS)
  AB = jnp.concatenate([ab_src, ab_dst], axis=1)
  AB2 = jnp.concatenate([ab_dst, ab_src], axis=1)
  perm16 = jnp.array([4, 5, 6, 7, 0, 1, 2, 3,
                      12, 13, 14, 15, 8, 9, 10, 11])
  ATT = jnp.concatenate([AB, AB[:, perm16], AB2, AB2[:, perm16]], axis=1)

  rows = jnp.arange(16)
  cols = jnp.arange(HID // 2)
  SL = (rows[:, None] == cols[None, :] // HI).astype(jnp.float32)

  G = jnp.zeros((OUT, 32), jnp.float32)
  G = G.at[:, :OUT].set(jnp.eye(OUT, dtype=jnp.float32))
  G = G.at[:, OUT:].set(jnp.broadcast_to(att_src2[0][:, None], (OUT, 16)))
  G2 = jnp.broadcast_to(att_dst2[0][:, None], (OUT, 16)).astype(jnp.float32)

  loops = jnp.arange(N, dtype=jnp.int32)
  pad = EPAD - ETOT
  srcs = jnp.concatenate([edge_index[0], loops,
                          jnp.zeros((pad,), jnp.int32)])
  dsts = jnp.concatenate([edge_index[1], loops,
                          jnp.full((pad,), N, jnp.int32)])
  edges = srcs | (dsts << 14)

  W1s = jnp.stack([W1[:, :HID // 2], W1[:, HID // 2:]])
  M = jnp.stack([W1 @ ATT[:, 0:16], W1 @ ATT[:, 16:32]])
  Nm = jnp.stack([W1 @ ATT[:, 32:48], W1 @ ATT[:, 48:64]])
  hstk, asd_stk, ads_stk = _proj1(x, W1s, M, Nm)
  accs, dens = _edge1(hstk, asd_stk, ads_stk, edges)
  hs2, ad2 = _comb1(accs[0, :N], accs[1, :N], dens[0, :N], dens[1, :N],
                    W2[:HID // 2], W2[HID // 2:], SL, SL, G, G2,
                    b1[:HID // 2].reshape(1, -1), b1[HID // 2:].reshape(1, -1))
  ad2p = jnp.concatenate([ad2, jnp.zeros((8, 16), jnp.float32)], axis=0)
  acc2, den2 = _edge2(hs2, ad2p, edges)
  out = _final(acc2[0, :N], acc2[1, :N], den2[0, :N], den2[1, :N],
               b2.reshape(1, OUT))
  return out
